# Initial kernel scaffold; baseline (speedup 1.0000x reference)
import functools

import jax
import jax.numpy as jnp
from jax import lax
from jax.experimental import pallas as pl
from jax.experimental.pallas import tpu as pltpu
from jax.experimental.pallas import tpu_sc as plsc

N = 10000
E = 320000
D = 128

_NC = 2
_NS = 16
_NW = _NC * _NS
_EPW = E // _NW
_C = 80
_NCHUNK = _EPW // _C
_NPW = N // _NS


def _proj_body(nodes_ref, w2_ref, w3_ref, p1_ref, p2_ref):
    n = nodes_ref[...]
    p1_ref[...] = jnp.dot(n, w2_ref[...], preferred_element_type=jnp.float32)
    p2_ref[...] = jnp.dot(n, w3_ref[...], preferred_element_type=jnp.float32)


def _edge_mm_body(e_ref, w_ref, b_ref, t_ref):
    t_ref[...] = (
        jnp.dot(e_ref[...], w_ref[...], preferred_element_type=jnp.float32)
        + b_ref[...]
    )


def _ln(y, g, b):
    mean = jnp.mean(y, axis=-1, keepdims=True)
    yc = y - mean
    var = jnp.mean(yc * yc, axis=-1, keepdims=True)
    return yc * lax.rsqrt(var + 1e-6) * g + b


def _edge_ln_body(ne_ref, e_ref, g_ref, b_ref, out_ref):
    out_ref[...] = _ln(ne_ref[...] + e_ref[...], g_ref[...], b_ref[...])


def _node_body(nodes_ref, r2_ref, wn1_ref, wn2_ref, bn_ref, g_ref, b_ref,
               out_ref):
    nodes = nodes_ref[...]
    r = r2_ref[:N, :] + r2_ref[N:, :]
    nn = (
        jnp.dot(nodes, wn1_ref[...], preferred_element_type=jnp.float32)
        + jnp.dot(r, wn2_ref[...], preferred_element_type=jnp.float32)
        + bn_ref[...]
    )
    out_ref[...] = _ln(nn + nodes, g_ref[...], b_ref[...])


def _edge_sc_body(t_hbm, p1_hbm, p2_hbm, s_hbm, r_hbm,
                  ne_hbm, r2_hbm,
                  s_v, r_v, t_v, p1_v, p2_v, zbuf, accum, gsem):
    cid = lax.axis_index("c")
    sid = lax.axis_index("s")
    wid = sid * _NC + cid

    zrows = _NPW // 5
    def zfill(i, _):
        for j in range(D // 16):
            zbuf[i, pl.ds(j * 16, 16)] = jnp.zeros((16,), jnp.float32)
        return 0
    lax.fori_loop(0, zrows, zfill, 0)
    row0 = sid * _NPW
    for k in range(5):
        pltpu.sync_copy(zbuf, accum.at[pl.ds(row0 + k * zrows, zrows)])
    plsc.subcore_barrier()

    def chunk(c, _):
        base = wid * _EPW + c * _C
        pltpu.sync_copy(s_hbm.at[pl.ds(base, _C)], s_v)
        pltpu.sync_copy(r_hbm.at[pl.ds(base, _C)], r_v)
        pltpu.sync_copy(t_hbm.at[pl.ds(base, _C)], t_v)
        pltpu.async_copy(p1_hbm.at[s_v], p1_v, gsem).wait()
        pltpu.async_copy(p2_hbm.at[r_v], p2_v, gsem).wait()

        def row(i, _):
            for j in range(D // 16):
                sl = pl.ds(j * 16, 16)
                t_v[i, sl] = t_v[i, sl] + p1_v[i, sl] + p2_v[i, sl]
            return 0
        lax.fori_loop(0, _C, row, 0)

        pltpu.sync_copy(t_v, ne_hbm.at[pl.ds(base, _C)])
        pltpu.sync_copy(t_v, accum.at[r_v], add=True)
        return 0
    lax.fori_loop(0, _NCHUNK, chunk, 0)

    plsc.subcore_barrier()
    pltpu.sync_copy(accum.at[pl.ds(row0, _NPW)],
                    r2_hbm.at[pl.ds(cid * N + row0, _NPW)])


@functools.partial(
    pl.kernel,
    out_type=[
        jax.ShapeDtypeStruct((E, D), jnp.float32),
        jax.ShapeDtypeStruct((2 * N, D), jnp.float32),
    ],
    mesh=plsc.VectorSubcoreMesh(core_axis_name="c", subcore_axis_name="s"),
    scratch_types=[
        pltpu.VMEM((_C,), jnp.int32),
        pltpu.VMEM((_C,), jnp.int32),
        pltpu.VMEM((_C, D), jnp.float32),
        pltpu.VMEM((_C, D), jnp.float32),
        pltpu.VMEM((_C, D), jnp.float32),
        pltpu.VMEM((N // _NS // 5, D), jnp.float32),
        pltpu.VMEM_SHARED((N, D), jnp.float32),
        pltpu.SemaphoreType.DMA,
    ],
)
def _edge_sc(*args):
    _edge_sc_body(*args)


def kernel(nodes, edges, senders, receivers, W_e, b_e, W_n, b_n,
           gamma_n, beta_n, gamma_e, beta_e):
    W1, W2, W3 = W_e[:D], W_e[D:2 * D], W_e[2 * D:]
    b_e2 = b_e.reshape(1, D)
    g_e2, bt_e2 = gamma_e.reshape(1, D), beta_e.reshape(1, D)
    g_n2, bt_n2 = gamma_n.reshape(1, D), beta_n.reshape(1, D)

    p1, p2 = pl.pallas_call(
        _proj_body,
        out_shape=[jax.ShapeDtypeStruct((N, D), jnp.float32)] * 2,
    )(nodes, W2, W3)

    BE = 4000
    t = pl.pallas_call(
        _edge_mm_body,
        grid=(E // BE,),
        in_specs=[
            pl.BlockSpec((BE, D), lambda i: (i, 0)),
            pl.BlockSpec((D, D), lambda i: (0, 0)),
            pl.BlockSpec((1, D), lambda i: (0, 0)),
        ],
        out_specs=pl.BlockSpec((BE, D), lambda i: (i, 0)),
        out_shape=jax.ShapeDtypeStruct((E, D), jnp.float32),
    )(edges, W1, b_e2)

    ne, r2 = _edge_sc(t, p1, p2, senders, receivers)

    edges_out = pl.pallas_call(
        _edge_ln_body,
        grid=(E // BE,),
        in_specs=[
            pl.BlockSpec((BE, D), lambda i: (i, 0)),
            pl.BlockSpec((BE, D), lambda i: (i, 0)),
            pl.BlockSpec((1, D), lambda i: (0, 0)),
            pl.BlockSpec((1, D), lambda i: (0, 0)),
        ],
        out_specs=pl.BlockSpec((BE, D), lambda i: (i, 0)),
        out_shape=jax.ShapeDtypeStruct((E, D), jnp.float32),
    )(ne, edges, g_e2, bt_e2)

    nodes_out = pl.pallas_call(
        _node_body,
        out_shape=jax.ShapeDtypeStruct((N, D), jnp.float32),
    )(nodes, r2, W_n[:D], W_n[D:], b_n.reshape(1, D), g_n2, bt_n2)

    return nodes_out, edges_out


# speedup vs baseline: 2.8213x; 2.8213x over previous
import functools

import jax
import jax.numpy as jnp
from jax import lax
from jax.experimental import pallas as pl
from jax.experimental.pallas import tpu as pltpu
from jax.experimental.pallas import tpu_sc as plsc

N = 10000
E = 320000
D = 128

_NC = 2
_NS = 16
_NW = _NC * _NS
_EPW = E // _NW
_C = 80
_NCHUNK = _EPW // _C
_NPW = N // _NS


def _proj_body(nodes_ref, w2_ref, w3_ref, p1_ref, p2_ref):
    n = nodes_ref[...]
    p1_ref[...] = jnp.dot(n, w2_ref[...], preferred_element_type=jnp.float32)
    p2_ref[...] = jnp.dot(n, w3_ref[...], preferred_element_type=jnp.float32)


def _edge_mm_body(e_ref, w_ref, b_ref, t_ref):
    t_ref[...] = (
        jnp.dot(e_ref[...], w_ref[...], preferred_element_type=jnp.float32)
        + b_ref[...]
    )


def _ln(y, g, b):
    mean = jnp.mean(y, axis=-1, keepdims=True)
    yc = y - mean
    var = jnp.mean(yc * yc, axis=-1, keepdims=True)
    return yc * lax.rsqrt(var + 1e-6) * g + b


def _edge_ln_body(ne_ref, e_ref, g_ref, b_ref, out_ref):
    out_ref[...] = _ln(ne_ref[...] + e_ref[...], g_ref[...], b_ref[...])


def _node_body(nodes_ref, r2_ref, wn1_ref, wn2_ref, bn_ref, g_ref, b_ref,
               out_ref):
    nodes = nodes_ref[...]
    r = r2_ref[:N, :] + r2_ref[N:, :]
    nn = (
        jnp.dot(nodes, wn1_ref[...], preferred_element_type=jnp.float32)
        + jnp.dot(r, wn2_ref[...], preferred_element_type=jnp.float32)
        + bn_ref[...]
    )
    out_ref[...] = _ln(nn + nodes, g_ref[...], b_ref[...])


def _edge_sc_body(t_hbm, p1_hbm, p2_hbm, s_hbm, r_hbm,
                  ne_hbm, r2_hbm,
                  s_v, r_v, t_v, p1_v, p2_v, zbuf, accum, gsem):
    cid = lax.axis_index("c")
    sid = lax.axis_index("s")
    wid = sid * _NC + cid

    zrows = _NPW // 5
    def zfill(i, _):
        for j in range(D // 16):
            zbuf[i, pl.ds(j * 16, 16)] = jnp.zeros((16,), jnp.float32)
        return 0
    lax.fori_loop(0, zrows, zfill, 0)
    row0 = sid * _NPW
    for k in range(5):
        pltpu.sync_copy(zbuf, accum.at[pl.ds(row0 + k * zrows, zrows)])
    plsc.subcore_barrier()

    def chunk(c, _):
        base = wid * _EPW + c * _C
        pltpu.sync_copy(s_hbm.at[pl.ds(base, _C)], s_v)
        pltpu.sync_copy(r_hbm.at[pl.ds(base, _C)], r_v)
        pltpu.sync_copy(t_hbm.at[pl.ds(base, _C)], t_v)
        pltpu.async_copy(p1_hbm.at[s_v], p1_v, gsem).wait()
        pltpu.async_copy(p2_hbm.at[r_v], p2_v, gsem).wait()

        def row(i, _):
            for j in range(D // 16):
                sl = pl.ds(j * 16, 16)
                t_v[i, sl] = t_v[i, sl] + p1_v[i, sl] + p2_v[i, sl]
            return 0
        lax.fori_loop(0, _C, row, 0)

        pltpu.sync_copy(t_v, ne_hbm.at[pl.ds(base, _C)])
        pltpu.sync_copy(t_v, accum.at[r_v], add=True)
        return 0
    lax.fori_loop(0, _NCHUNK, chunk, 0)

    plsc.subcore_barrier()
    o = sid * 624

    @pl.when(sid < _NS - 1)
    def _copy_main():
        pltpu.sync_copy(accum.at[pl.ds(o, 624)],
                        r2_hbm.at[pl.ds(cid * N + o, 624)])

    @pl.when(sid == _NS - 1)
    def _copy_tail():
        pltpu.sync_copy(accum.at[pl.ds(o, 640)],
                        r2_hbm.at[pl.ds(cid * N + o, 640)])


@functools.partial(
    pl.kernel,
    out_type=[
        jax.ShapeDtypeStruct((E, D), jnp.float32),
        jax.ShapeDtypeStruct((2 * N, D), jnp.float32),
    ],
    mesh=plsc.VectorSubcoreMesh(core_axis_name="c", subcore_axis_name="s"),
    scratch_types=[
        pltpu.VMEM((_C,), jnp.int32),
        pltpu.VMEM((_C,), jnp.int32),
        pltpu.VMEM((_C, D), jnp.float32),
        pltpu.VMEM((_C, D), jnp.float32),
        pltpu.VMEM((_C, D), jnp.float32),
        pltpu.VMEM((N // _NS // 5, D), jnp.float32),
        pltpu.VMEM_SHARED((N, D), jnp.float32),
        pltpu.SemaphoreType.DMA,
    ],
)
def _edge_sc(*args):
    _edge_sc_body(*args)


def kernel(nodes, edges, senders, receivers, W_e, b_e, W_n, b_n,
           gamma_n, beta_n, gamma_e, beta_e):
    W1, W2, W3 = W_e[:D], W_e[D:2 * D], W_e[2 * D:]
    b_e2 = b_e.reshape(1, D)
    g_e2, bt_e2 = gamma_e.reshape(1, D), beta_e.reshape(1, D)
    g_n2, bt_n2 = gamma_n.reshape(1, D), beta_n.reshape(1, D)

    p1, p2 = pl.pallas_call(
        _proj_body,
        out_shape=[jax.ShapeDtypeStruct((N, D), jnp.float32)] * 2,
    )(nodes, W2, W3)

    BE = 4000
    t = pl.pallas_call(
        _edge_mm_body,
        grid=(E // BE,),
        in_specs=[
            pl.BlockSpec((BE, D), lambda i: (i, 0)),
            pl.BlockSpec((D, D), lambda i: (0, 0)),
            pl.BlockSpec((1, D), lambda i: (0, 0)),
        ],
        out_specs=pl.BlockSpec((BE, D), lambda i: (i, 0)),
        out_shape=jax.ShapeDtypeStruct((E, D), jnp.float32),
    )(edges, W1, b_e2)

    ne, r2 = _edge_sc(t, p1, p2, senders, receivers)

    edges_out = pl.pallas_call(
        _edge_ln_body,
        grid=(E // BE,),
        in_specs=[
            pl.BlockSpec((BE, D), lambda i: (i, 0)),
            pl.BlockSpec((BE, D), lambda i: (i, 0)),
            pl.BlockSpec((1, D), lambda i: (0, 0)),
            pl.BlockSpec((1, D), lambda i: (0, 0)),
        ],
        out_specs=pl.BlockSpec((BE, D), lambda i: (i, 0)),
        out_shape=jax.ShapeDtypeStruct((E, D), jnp.float32),
    )(ne, edges, g_e2, bt_e2)

    nodes_out = pl.pallas_call(
        _node_body,
        out_shape=jax.ShapeDtypeStruct((N, D), jnp.float32),
    )(nodes, r2, W_n[:D], W_n[D:], b_n.reshape(1, D), g_n2, bt_n2)

    return nodes_out, edges_out


# speedup vs baseline: 4.1408x; 1.4677x over previous
import functools

import jax
import jax.numpy as jnp
from jax import lax
from jax.experimental import pallas as pl
from jax.experimental.pallas import tpu as pltpu
from jax.experimental.pallas import tpu_sc as plsc

N = 10000
E = 320000
D = 128

_NC = 2
_NS = 16
_NW = _NC * _NS
_EPW = E // _NW
_C = 16
_NCHUNK = _EPW // _C
_NB = 5
_NPW = N // _NS


def _proj_body(nodes_ref, w2_ref, w3_ref, be_ref, p1_ref, p2_ref):
    n = nodes_ref[...]
    p1_ref[...] = jnp.dot(n, w2_ref[...], preferred_element_type=jnp.float32)
    p2_ref[...] = (
        jnp.dot(n, w3_ref[...], preferred_element_type=jnp.float32)
        + be_ref[...]
    )


def _ln(y, g, b):
    mean = jnp.mean(y, axis=-1, keepdims=True)
    yc = y - mean
    var = jnp.mean(yc * yc, axis=-1, keepdims=True)
    return yc * lax.rsqrt(var + 1e-6) * g + b


def _edge_ep_body(e_ref, gp_ref, w1_ref, g_ref, b_ref, out_ref):
    e = e_ref[...]
    ne = (
        jnp.dot(e, w1_ref[...], preferred_element_type=jnp.float32)
        + gp_ref[...]
    )
    out_ref[...] = _ln(ne + e, g_ref[...], b_ref[...])


def _node_body(nodes_ref, se_ref, rg_ref, w1_ref, wn1_ref, wn2_ref, bn_ref,
               g_ref, b_ref, out_ref):
    nodes = nodes_ref[...]
    se = se_ref[:N, :] + se_ref[N:, :]
    r = (
        jnp.dot(se, w1_ref[...], preferred_element_type=jnp.float32)
        + rg_ref[:N, :] + rg_ref[N:, :]
    )
    nn = (
        jnp.dot(nodes, wn1_ref[...], preferred_element_type=jnp.float32)
        + jnp.dot(r, wn2_ref[...], preferred_element_type=jnp.float32)
        + bn_ref[...]
    )
    out_ref[...] = _ln(nn + nodes, g_ref[...], b_ref[...])


def _edge_sc_body(edges_hbm, p1_hbm, p2_hbm, s_hbm, r_hbm,
                  g_hbm, se_hbm, rg_hbm,
                  s_v, r_v, p1_v, p2_v, accum,
                  lsem, gsem, wsem, asem):
    cid = lax.axis_index("c")
    sid = lax.axis_index("s")
    wid = sid * _NC + cid
    row0 = sid * _NPW

    def _zero_accum():
        def zfill(i, _):
            for g in range(D // 16):
                p1_v[0, i, pl.ds(g * 16, 16)] = jnp.zeros((16,), jnp.float32)
            return 0
        lax.fori_loop(0, _C, zfill, 0)

        def zcopy(k, _):
            pltpu.sync_copy(p1_v.at[0], accum.at[pl.ds(row0 + k * _C, _C)])
            return 0
        lax.fori_loop(0, _NPW // _C, zcopy, 0)
        pltpu.sync_copy(p1_v.at[0, pl.ds(0, 1)],
                        accum.at[pl.ds(row0 + (_NPW // _C) * _C, 1)])

    def _dump_accum(dst_hbm):
        o = sid * 624

        @pl.when(sid < _NS - 1)
        def _():
            pltpu.sync_copy(accum.at[pl.ds(o, 624)],
                            dst_hbm.at[pl.ds(cid * N + o, 624)])

        @pl.when(sid == _NS - 1)
        def _():
            pltpu.sync_copy(accum.at[pl.ds(o, 640)],
                            dst_hbm.at[pl.ds(cid * N + o, 640)])

    def _base(c):
        return wid * _EPW + c * _C

    _zero_accum()
    plsc.subcore_barrier()

    def p1_issue_loads(c, j):
        b = _base(c)
        pltpu.async_copy(r_hbm.at[pl.ds(b, _C)], r_v.at[j], lsem.at[j])
        pltpu.async_copy(edges_hbm.at[pl.ds(b, _C)], p1_v.at[j], lsem.at[j])

    def p1_wait_loads(c, j):
        b = _base(c)
        pltpu.make_async_copy(r_hbm.at[pl.ds(b, _C)], r_v.at[j],
                              lsem.at[j]).wait()
        pltpu.make_async_copy(edges_hbm.at[pl.ds(b, _C)], p1_v.at[j],
                              lsem.at[j]).wait()

    def p1_issue_scatter(j):
        pltpu.async_copy(p1_v.at[j], accum.at[r_v.at[j]], asem.at[j],
                         add=True)

    def p1_wait_scatter(j):
        pltpu.make_async_copy(p1_v.at[j], accum.at[r_v.at[j]],
                              asem.at[j]).wait()

    p1_issue_loads(0, 0)
    p1_issue_loads(1, 1)

    def p1_outer(k, _):
        for jj in range(_NB):
            c = k * _NB + jj
            j = jj
            jl = (jj + 2) % _NB

            @pl.when(c >= 3)
            def _():
                p1_wait_scatter(jl)

            @pl.when(c + 2 < _NCHUNK)
            def _():
                p1_issue_loads(c + 2, jl)

            p1_wait_loads(c, j)
            p1_issue_scatter(j)
        return 0
    lax.fori_loop(0, _NCHUNK // _NB, p1_outer, 0)
    for c in (_NCHUNK - 3, _NCHUNK - 2, _NCHUNK - 1):
        p1_wait_scatter(c % _NB)

    plsc.subcore_barrier()
    _dump_accum(se_hbm)
    plsc.subcore_barrier()

    _zero_accum()
    plsc.subcore_barrier()

    def p2_issue_loads(c, j):
        b = _base(c)
        pltpu.async_copy(s_hbm.at[pl.ds(b, _C)], s_v.at[j], lsem.at[j])
        pltpu.async_copy(r_hbm.at[pl.ds(b, _C)], r_v.at[j], lsem.at[j])

    def p2_wait_loads(c, j):
        b = _base(c)
        pltpu.make_async_copy(s_hbm.at[pl.ds(b, _C)], s_v.at[j],
                              lsem.at[j]).wait()
        pltpu.make_async_copy(r_hbm.at[pl.ds(b, _C)], r_v.at[j],
                              lsem.at[j]).wait()

    def p2_issue_gathers(j):
        pltpu.async_copy(p1_hbm.at[s_v.at[j]], p1_v.at[j], gsem.at[j])
        pltpu.async_copy(p2_hbm.at[r_v.at[j]], p2_v.at[j], gsem.at[j])

    def p2_wait_gathers(j):
        pltpu.make_async_copy(p1_hbm.at[s_v.at[j]], p1_v.at[j],
                              gsem.at[j]).wait()
        pltpu.make_async_copy(p2_hbm.at[r_v.at[j]], p2_v.at[j],
                              gsem.at[j]).wait()

    def p2_issue_writes(c, j):
        b = _base(c)
        pltpu.async_copy(p1_v.at[j], g_hbm.at[pl.ds(b, _C)], wsem.at[j])
        pltpu.async_copy(p1_v.at[j], accum.at[r_v.at[j]], asem.at[j],
                         add=True)

    def p2_wait_writes(c, j):
        b = _base(c)
        pltpu.make_async_copy(p1_v.at[j], g_hbm.at[pl.ds(b, _C)],
                              wsem.at[j]).wait()
        pltpu.make_async_copy(p1_v.at[j], accum.at[r_v.at[j]],
                              asem.at[j]).wait()

    p2_issue_loads(0, 0)
    p2_issue_loads(1, 1)
    p2_wait_loads(0, 0)
    p2_issue_gathers(0)

    def p2_outer(k, _):
        for jj in range(_NB):
            c = k * _NB + jj
            j = jj
            jn = (jj + 1) % _NB
            jl = (jj + 2) % _NB

            @pl.when(c >= 3)
            def _():
                p2_wait_writes(c - 3, jl)

            @pl.when(c + 2 < _NCHUNK)
            def _():
                p2_issue_loads(c + 2, jl)

            @pl.when(c + 1 < _NCHUNK)
            def _():
                p2_wait_loads(c + 1, jn)
                p2_issue_gathers(jn)

            p2_wait_gathers(j)

            p1s, p2s = p1_v.at[j], p2_v.at[j]

            def row(i, _):
                for g in range(D // 16):
                    sl = pl.ds(g * 16, 16)
                    p1s[i, sl] = p1s[i, sl] + p2s[i, sl]
                return 0
            lax.fori_loop(0, _C, row, 0)

            p2_issue_writes(c, j)
        return 0
    lax.fori_loop(0, _NCHUNK // _NB, p2_outer, 0)
    for c in (_NCHUNK - 3, _NCHUNK - 2, _NCHUNK - 1):
        p2_wait_writes(c, c % _NB)

    plsc.subcore_barrier()
    _dump_accum(rg_hbm)


@functools.partial(
    pl.kernel,
    out_type=[
        jax.ShapeDtypeStruct((E, D), jnp.float32),
        jax.ShapeDtypeStruct((2 * N, D), jnp.float32),
        jax.ShapeDtypeStruct((2 * N, D), jnp.float32),
    ],
    mesh=plsc.VectorSubcoreMesh(core_axis_name="c", subcore_axis_name="s"),
    scratch_types=[
        pltpu.VMEM((_NB, _C), jnp.int32),
        pltpu.VMEM((_NB, _C), jnp.int32),
        pltpu.VMEM((_NB, _C, D), jnp.float32),
        pltpu.VMEM((_NB, _C, D), jnp.float32),
        pltpu.VMEM_SHARED((N, D), jnp.float32),
        pltpu.SemaphoreType.DMA((_NB,)),
        pltpu.SemaphoreType.DMA((_NB,)),
        pltpu.SemaphoreType.DMA((_NB,)),
        pltpu.SemaphoreType.DMA((_NB,)),
    ],
)
def _edge_sc(*args):
    _edge_sc_body(*args)


def kernel(nodes, edges, senders, receivers, W_e, b_e, W_n, b_n,
           gamma_n, beta_n, gamma_e, beta_e):
    W1, W2, W3 = W_e[:D], W_e[D:2 * D], W_e[2 * D:]
    b_e2 = b_e.reshape(1, D)
    g_e2, bt_e2 = gamma_e.reshape(1, D), beta_e.reshape(1, D)
    g_n2, bt_n2 = gamma_n.reshape(1, D), beta_n.reshape(1, D)

    p1, p2 = pl.pallas_call(
        _proj_body,
        out_shape=[jax.ShapeDtypeStruct((N, D), jnp.float32)] * 2,
    )(nodes, W2, W3, b_e2)

    gp, se2, rg2 = _edge_sc(edges, p1, p2, senders, receivers)

    BE = 4000
    edges_out = pl.pallas_call(
        _edge_ep_body,
        grid=(E // BE,),
        in_specs=[
            pl.BlockSpec((BE, D), lambda i: (i, 0)),
            pl.BlockSpec((BE, D), lambda i: (i, 0)),
            pl.BlockSpec((D, D), lambda i: (0, 0)),
            pl.BlockSpec((1, D), lambda i: (0, 0)),
            pl.BlockSpec((1, D), lambda i: (0, 0)),
        ],
        out_specs=pl.BlockSpec((BE, D), lambda i: (i, 0)),
        out_shape=jax.ShapeDtypeStruct((E, D), jnp.float32),
    )(edges, gp, W1, g_e2, bt_e2)

    nodes_out = pl.pallas_call(
        _node_body,
        out_shape=jax.ShapeDtypeStruct((N, D), jnp.float32),
    )(nodes, se2, rg2, W1, W_n[:D], W_n[D:], b_n.reshape(1, D), g_n2, bt_n2)

    return nodes_out, edges_out


# speedup vs baseline: 4.6722x; 1.1283x over previous
import dataclasses
import functools

import jax
import jax.numpy as jnp
from jax import lax
from jax.experimental import pallas as pl
from jax.experimental.pallas import tpu as pltpu
from jax.experimental.pallas import tpu_sc as plsc

N = 10000
E = 320000
D = 128

_NC = 2
_NS = 16
_NW = _NC * _NS
_EPW = E // _NW
_C = 16
_NCHUNK = _EPW // _C
_NB = 5
_NPW = N // _NS


def _proj_body(nodes_ref, w2_ref, w3_ref, be_ref, p1_ref, p2_ref):
    n = nodes_ref[...]
    p1_ref[...] = jnp.dot(
        n, w2_ref[...], preferred_element_type=jnp.float32
    )
    p2_ref[...] = (
        jnp.dot(n, w3_ref[...], preferred_element_type=jnp.float32)
        + be_ref[...]
    )


def _ln(y, g, b):
    mean = jnp.mean(y, axis=-1, keepdims=True)
    yc = y - mean
    var = jnp.mean(yc * yc, axis=-1, keepdims=True)
    return yc * lax.rsqrt(var + 1e-6) * g + b


def _edge_ep_body(e_ref, gp_ref, w1_ref, g_ref, b_ref, out_ref):
    e = e_ref[...]
    ne = (
        jnp.dot(e, w1_ref[...], preferred_element_type=jnp.float32)
        + gp_ref[...]
    )
    out_ref[...] = _ln(ne + e, g_ref[...], b_ref[...])


def _node_body(nodes_ref, se_ref, rg_ref, w1_ref, wn1_ref, wn2_ref, bn_ref,
               g_ref, b_ref, out_ref):
    nodes = nodes_ref[...]
    se = se_ref[:N, :] + se_ref[N:, :]
    r = (
        jnp.dot(se, w1_ref[...], preferred_element_type=jnp.float32)
        + rg_ref[:N, :] + rg_ref[N:, :]
    )
    nn = (
        jnp.dot(nodes, wn1_ref[...], preferred_element_type=jnp.float32)
        + jnp.dot(r, wn2_ref[...], preferred_element_type=jnp.float32)
        + bn_ref[...]
    )
    out_ref[...] = _ln(nn + nodes, g_ref[...], b_ref[...])


def _edge_sc_body(edges_hbm, p1_hbm, p2_hbm, s_hbm, r_hbm,
                  g_hbm, se_hbm, rg_hbm,
                  s_v, r_v, p1_v, p2_v, accum,
                  lsem, gsem, wsem, asem):
    cid = lax.axis_index("c")
    sid = lax.axis_index("s")
    wid = sid * _NC + cid
    row0 = sid * _NPW

    def _zero_accum():
        def zfill(i, _):
            for g in range(D // 16):
                p1_v[0, i, pl.ds(g * 16, 16)] = jnp.zeros((16,), jnp.float32)
            return 0
        lax.fori_loop(0, _C, zfill, 0)

        def zcopy(k, _):
            pltpu.sync_copy(p1_v.at[0], accum.at[pl.ds(row0 + k * _C, _C)])
            return 0
        lax.fori_loop(0, _NPW // _C, zcopy, 0)
        pltpu.sync_copy(p1_v.at[0, pl.ds(0, 1)],
                        accum.at[pl.ds(row0 + (_NPW // _C) * _C, 1)])

    def _dump_accum(dst_hbm):
        o = sid * 624

        @pl.when(sid < _NS - 1)
        def _():
            pltpu.sync_copy(accum.at[pl.ds(o, 624)],
                            dst_hbm.at[pl.ds(cid * N + o, 624)])

        @pl.when(sid == _NS - 1)
        def _():
            pltpu.sync_copy(accum.at[pl.ds(o, 640)],
                            dst_hbm.at[pl.ds(cid * N + o, 640)])

    def _base(c):
        return wid * _EPW + c * _C

    _zero_accum()
    plsc.subcore_barrier()

    def p1_issue_loads(c, j):
        b = _base(c)
        pltpu.async_copy(r_hbm.at[pl.ds(b, _C)], r_v.at[j], lsem.at[j])
        pltpu.async_copy(edges_hbm.at[pl.ds(b, _C)], p1_v.at[j], lsem.at[j])

    def p1_wait_loads(c, j):
        b = _base(c)
        pltpu.make_async_copy(r_hbm.at[pl.ds(b, _C)], r_v.at[j],
                              lsem.at[j]).wait()
        pltpu.make_async_copy(edges_hbm.at[pl.ds(b, _C)], p1_v.at[j],
                              lsem.at[j]).wait()

    def p1_issue_scatter(j):
        pltpu.async_copy(p1_v.at[j], accum.at[r_v.at[j]], asem.at[j],
                         add=True)

    def p1_wait_scatter(j):
        pltpu.make_async_copy(p1_v.at[j], accum.at[r_v.at[j]],
                              asem.at[j]).wait()

    p1_issue_loads(0, 0)
    p1_issue_loads(1, 1)

    def p1_outer(k, _):
        for jj in range(_NB):
            c = k * _NB + jj
            j = jj
            jl = (jj + 2) % _NB

            @pl.when(c >= 3)
            def _():
                p1_wait_scatter(jl)

            @pl.when(c + 2 < _NCHUNK)
            def _():
                p1_issue_loads(c + 2, jl)

            p1_wait_loads(c, j)
            p1_issue_scatter(j)
        return 0
    lax.fori_loop(0, _NCHUNK // _NB, p1_outer, 0)
    for c in (_NCHUNK - 3, _NCHUNK - 2, _NCHUNK - 1):
        p1_wait_scatter(c % _NB)

    plsc.subcore_barrier()
    _dump_accum(se_hbm)
    plsc.subcore_barrier()

    _zero_accum()
    plsc.subcore_barrier()

    def p2_issue_loads(c, j):
        b = _base(c)
        pltpu.async_copy(s_hbm.at[pl.ds(b, _C)], s_v.at[j], lsem.at[j])
        pltpu.async_copy(r_hbm.at[pl.ds(b, _C)], r_v.at[j], lsem.at[j])

    def p2_wait_loads(c, j):
        b = _base(c)
        pltpu.make_async_copy(s_hbm.at[pl.ds(b, _C)], s_v.at[j],
                              lsem.at[j]).wait()
        pltpu.make_async_copy(r_hbm.at[pl.ds(b, _C)], r_v.at[j],
                              lsem.at[j]).wait()

    def p2_issue_gathers(j):
        pltpu.async_copy(p1_hbm.at[s_v.at[j]], p1_v.at[j], gsem.at[j])
        pltpu.async_copy(p2_hbm.at[r_v.at[j]], p2_v.at[j], gsem.at[j])

    def p2_wait_gathers(j):
        pltpu.make_async_copy(p1_hbm.at[s_v.at[j]], p1_v.at[j],
                              gsem.at[j]).wait()
        pltpu.make_async_copy(p2_hbm.at[r_v.at[j]], p2_v.at[j],
                              gsem.at[j]).wait()

    def p2_issue_writes(c, j):
        b = _base(c)
        pltpu.async_copy(p1_v.at[j], g_hbm.at[pl.ds(b, _C)], wsem.at[j])
        pltpu.async_copy(p1_v.at[j], accum.at[r_v.at[j]], asem.at[j],
                         add=True)

    def p2_wait_writes(c, j):
        b = _base(c)
        pltpu.make_async_copy(p1_v.at[j], g_hbm.at[pl.ds(b, _C)],
                              wsem.at[j]).wait()
        pltpu.make_async_copy(p1_v.at[j], accum.at[r_v.at[j]],
                              asem.at[j]).wait()

    p2_issue_loads(0, 0)
    p2_issue_loads(1, 1)
    p2_issue_loads(2, 2)
    p2_wait_loads(0, 0)
    p2_issue_gathers(0)
    p2_wait_loads(1, 1)
    p2_issue_gathers(1)

    def p2_outer(k, _):
        for jj in range(_NB):
            c = k * _NB + jj
            j = jj
            jg = (jj + 2) % _NB
            jl = (jj + 3) % _NB

            @pl.when(c >= 2)
            def _():
                p2_wait_writes(c - 2, jl)

            @pl.when(c + 3 < _NCHUNK)
            def _():
                p2_issue_loads(c + 3, jl)

            @pl.when(c + 2 < _NCHUNK)
            def _():
                p2_wait_loads(c + 2, jg)
                p2_issue_gathers(jg)

            p2_wait_gathers(j)

            p1s, p2s = p1_v.at[j], p2_v.at[j]

            def row(i, _):
                for g in range(D // 16):
                    sl = pl.ds(g * 16, 16)
                    p1s[i, sl] = p1s[i, sl] + p2s[i, sl]
                return 0
            lax.fori_loop(0, _C, row, 0)

            p2_issue_writes(c, j)
        return 0
    lax.fori_loop(0, _NCHUNK // _NB, p2_outer, 0)
    for c in (_NCHUNK - 2, _NCHUNK - 1):
        p2_wait_writes(c, c % _NB)

    plsc.subcore_barrier()
    _dump_accum(rg_hbm)


_SC_CP = pltpu.CompilerParams()
if "needs_layout_passes" in pltpu.CompilerParams.__dataclass_fields__:
    _SC_CP = dataclasses.replace(_SC_CP, needs_layout_passes=False)


@functools.partial(
    pl.kernel,
    compiler_params=_SC_CP,
    out_type=[
        jax.ShapeDtypeStruct((E, D), jnp.float32),
        jax.ShapeDtypeStruct((2 * N, D), jnp.float32),
        jax.ShapeDtypeStruct((2 * N, D), jnp.float32),
    ],
    mesh=plsc.VectorSubcoreMesh(core_axis_name="c", subcore_axis_name="s"),
    scratch_types=[
        pltpu.VMEM((_NB, _C), jnp.int32),
        pltpu.VMEM((_NB, _C), jnp.int32),
        pltpu.VMEM((_NB, _C, D), jnp.float32),
        pltpu.VMEM((_NB, _C, D), jnp.float32),
        pltpu.VMEM_SHARED((N, D), jnp.float32),
        pltpu.SemaphoreType.DMA((_NB,)),
        pltpu.SemaphoreType.DMA((_NB,)),
        pltpu.SemaphoreType.DMA((_NB,)),
        pltpu.SemaphoreType.DMA((_NB,)),
    ],
)
def _edge_sc(*args):
    _edge_sc_body(*args)


def kernel(nodes, edges, senders, receivers, W_e, b_e, W_n, b_n,
           gamma_n, beta_n, gamma_e, beta_e):
    W1, W2, W3 = W_e[:D], W_e[D:2 * D], W_e[2 * D:]
    b_e2 = b_e.reshape(1, D)
    g_e2, bt_e2 = gamma_e.reshape(1, D), beta_e.reshape(1, D)
    g_n2, bt_n2 = gamma_n.reshape(1, D), beta_n.reshape(1, D)

    p1, p2 = pl.pallas_call(
        _proj_body,
        out_shape=[jax.ShapeDtypeStruct((N, D), jnp.float32)] * 2,
    )(nodes, W2, W3, b_e2)

    gp, se2, rg2 = _edge_sc(edges, p1, p2, senders, receivers)

    BE = 4000
    edges_out = pl.pallas_call(
        _edge_ep_body,
        grid=(E // BE,),
        in_specs=[
            pl.BlockSpec((BE, D), lambda i: (i, 0)),
            pl.BlockSpec((BE, D), lambda i: (i, 0)),
            pl.BlockSpec((D, D), lambda i: (0, 0)),
            pl.BlockSpec((1, D), lambda i: (0, 0)),
            pl.BlockSpec((1, D), lambda i: (0, 0)),
        ],
        out_specs=pl.BlockSpec((BE, D), lambda i: (i, 0)),
        out_shape=jax.ShapeDtypeStruct((E, D), jnp.float32),
    )(edges, gp, W1, g_e2, bt_e2)

    nodes_out = pl.pallas_call(
        _node_body,
        out_shape=jax.ShapeDtypeStruct((N, D), jnp.float32),
    )(nodes, se2, rg2, W1, W_n[:D], W_n[D:], b_n.reshape(1, D), g_n2, bt_n2)

    return nodes_out, edges_out


# speedup vs baseline: 5.4307x; 1.1623x over previous
import dataclasses
import functools

import jax
import jax.numpy as jnp
from jax import lax
from jax.experimental import pallas as pl
from jax.experimental.pallas import tpu as pltpu
from jax.experimental.pallas import tpu_sc as plsc

N = 10000
E = 320000
D = 128

_NC = 2
_NS = 16
_NW = _NC * _NS
_EPW = E // _NW
_C = 16
_NCHUNK = _EPW // _C
_NB = 5
_NPW = N // _NS


def _proj_body(nodes_ref, w2_ref, w3_ref, be_ref, p1_ref, p2_ref):
    n = nodes_ref[...]
    p1_ref[...] = jnp.dot(
        n, w2_ref[...], preferred_element_type=jnp.float32
    )
    p2_ref[...] = (
        jnp.dot(n, w3_ref[...], preferred_element_type=jnp.float32)
        + be_ref[...]
    )


def _ln(y, g, b):
    mean = jnp.mean(y, axis=-1, keepdims=True)
    yc = y - mean
    var = jnp.mean(yc * yc, axis=-1, keepdims=True)
    return yc * lax.rsqrt(var + 1e-6) * g + b


def _edge_ep_body(e_ref, gp_ref, w1_ref, g_ref, b_ref, out_ref):
    e = e_ref[...]
    ne = (
        jnp.dot(e, w1_ref[...], preferred_element_type=jnp.float32)
        + gp_ref[...]
    )
    out_ref[...] = _ln(ne + e, g_ref[...], b_ref[...])


def _node_body(nodes_ref, se_ref, rg_ref, w1_ref, wn1_ref, wn2_ref, bn_ref,
               g_ref, b_ref, out_ref):
    nodes = nodes_ref[...]
    se = se_ref[:N, :] + se_ref[N:, :]
    r = (
        jnp.dot(se, w1_ref[...], preferred_element_type=jnp.float32)
        + rg_ref[:N, :] + rg_ref[N:, :]
    )
    nn = (
        jnp.dot(nodes, wn1_ref[...], preferred_element_type=jnp.float32)
        + jnp.dot(r, wn2_ref[...], preferred_element_type=jnp.float32)
        + bn_ref[...]
    )
    out_ref[...] = _ln(nn + nodes, g_ref[...], b_ref[...])


def _zero_accum(p1_v, accum, row0):
    def zfill(i, _):
        for g in range(D // 16):
            p1_v[0, i, pl.ds(g * 16, 16)] = jnp.zeros((16,), jnp.float32)
        return 0
    lax.fori_loop(0, _C, zfill, 0)

    def zcopy(k, _):
        pltpu.sync_copy(p1_v.at[0], accum.at[pl.ds(row0 + k * _C, _C)])
        return 0
    lax.fori_loop(0, _NPW // _C, zcopy, 0)
    pltpu.sync_copy(p1_v.at[0, pl.ds(0, 1)],
                    accum.at[pl.ds(row0 + (_NPW // _C) * _C, 1)])


def _dump_accum(accum, dst_hbm, sid, cid):
    o = sid * 624

    @pl.when(sid < _NS - 1)
    def _():
        pltpu.sync_copy(accum.at[pl.ds(o, 624)],
                        dst_hbm.at[pl.ds(cid * N + o, 624)])

    @pl.when(sid == _NS - 1)
    def _():
        pltpu.sync_copy(accum.at[pl.ds(o, 640)],
                        dst_hbm.at[pl.ds(cid * N + o, 640)])


def _sc_gather_body(p1_hbm, p2_hbm, s_hbm, r_hbm,
                    g_hbm, rg_hbm,
                    s_v, r_v, p1_v, p2_v, accum,
                    lsem, gsem, wsem, asem):
    cid = lax.axis_index("c")
    sid = lax.axis_index("s")
    wid = sid * _NC + cid
    row0 = sid * _NPW

    _zero_accum(p1_v, accum, row0)
    plsc.subcore_barrier()

    def _base(c):
        return wid * _EPW + c * _C

    def issue_loads(c, j):
        b = _base(c)
        pltpu.async_copy(s_hbm.at[pl.ds(b, _C)], s_v.at[j], lsem.at[j])
        pltpu.async_copy(r_hbm.at[pl.ds(b, _C)], r_v.at[j], lsem.at[j])

    def wait_loads(c, j):
        b = _base(c)
        pltpu.make_async_copy(s_hbm.at[pl.ds(b, _C)], s_v.at[j],
                              lsem.at[j]).wait()
        pltpu.make_async_copy(r_hbm.at[pl.ds(b, _C)], r_v.at[j],
                              lsem.at[j]).wait()

    def issue_gathers(j):
        pltpu.async_copy(p1_hbm.at[s_v.at[j]], p1_v.at[j], gsem.at[j])
        pltpu.async_copy(p2_hbm.at[r_v.at[j]], p2_v.at[j], gsem.at[j])

    def wait_gathers(j):
        pltpu.make_async_copy(p1_hbm.at[s_v.at[j]], p1_v.at[j],
                              gsem.at[j]).wait()
        pltpu.make_async_copy(p2_hbm.at[r_v.at[j]], p2_v.at[j],
                              gsem.at[j]).wait()

    def issue_writes(c, j):
        b = _base(c)
        pltpu.async_copy(p1_v.at[j], g_hbm.at[pl.ds(b, _C)], wsem.at[j])
        pltpu.async_copy(p1_v.at[j], accum.at[r_v.at[j]], asem.at[j],
                         add=True)

    def wait_writes(c, j):
        b = _base(c)
        pltpu.make_async_copy(p1_v.at[j], g_hbm.at[pl.ds(b, _C)],
                              wsem.at[j]).wait()
        pltpu.make_async_copy(p1_v.at[j], accum.at[r_v.at[j]],
                              asem.at[j]).wait()

    issue_loads(0, 0)
    issue_loads(1, 1)
    issue_loads(2, 2)
    wait_loads(0, 0)
    issue_gathers(0)
    wait_loads(1, 1)
    issue_gathers(1)

    def outer(k, _):
        for jj in range(_NB):
            c = k * _NB + jj
            j = jj
            jg = (jj + 2) % _NB
            jl = (jj + 3) % _NB

            @pl.when(c >= 2)
            def _():
                wait_writes(c - 2, jl)

            @pl.when(c + 3 < _NCHUNK)
            def _():
                issue_loads(c + 3, jl)

            @pl.when(c + 2 < _NCHUNK)
            def _():
                wait_loads(c + 2, jg)
                issue_gathers(jg)

            wait_gathers(j)

            p1s, p2s = p1_v.at[j], p2_v.at[j]

            def row(i, _):
                for g in range(D // 16):
                    sl = pl.ds(g * 16, 16)
                    p1s[i, sl] = p1s[i, sl] + p2s[i, sl]
                return 0
            lax.fori_loop(0, _C, row, 0)

            issue_writes(c, j)
        return 0
    lax.fori_loop(0, _NCHUNK // _NB, outer, 0)
    for c in (_NCHUNK - 2, _NCHUNK - 1):
        wait_writes(c, c % _NB)

    plsc.subcore_barrier()
    _dump_accum(accum, rg_hbm, sid, cid)


def _sc_segsum_body(edges_hbm, r_hbm,
                    se_hbm,
                    r_v, p1_v, accum, lsem, asem):
    cid = lax.axis_index("c")
    sid = lax.axis_index("s")
    wid = sid * _NC + cid
    row0 = sid * _NPW

    _zero_accum(p1_v, accum, row0)
    plsc.subcore_barrier()

    def _base(c):
        return wid * _EPW + c * _C

    def issue_loads(c, j):
        b = _base(c)
        pltpu.async_copy(r_hbm.at[pl.ds(b, _C)], r_v.at[j], lsem.at[j])
        pltpu.async_copy(edges_hbm.at[pl.ds(b, _C)], p1_v.at[j], lsem.at[j])

    def wait_loads(c, j):
        b = _base(c)
        pltpu.make_async_copy(r_hbm.at[pl.ds(b, _C)], r_v.at[j],
                              lsem.at[j]).wait()
        pltpu.make_async_copy(edges_hbm.at[pl.ds(b, _C)], p1_v.at[j],
                              lsem.at[j]).wait()

    def issue_scatter(j):
        pltpu.async_copy(p1_v.at[j], accum.at[r_v.at[j]], asem.at[j],
                         add=True)

    def wait_scatter(j):
        pltpu.make_async_copy(p1_v.at[j], accum.at[r_v.at[j]],
                              asem.at[j]).wait()

    issue_loads(0, 0)
    issue_loads(1, 1)

    def outer(k, _):
        for jj in range(_NB):
            c = k * _NB + jj
            j = jj
            jl = (jj + 2) % _NB

            @pl.when(c >= 3)
            def _():
                wait_scatter(jl)

            @pl.when(c + 2 < _NCHUNK)
            def _():
                issue_loads(c + 2, jl)

            wait_loads(c, j)
            issue_scatter(j)
        return 0
    lax.fori_loop(0, _NCHUNK // _NB, outer, 0)
    for c in (_NCHUNK - 3, _NCHUNK - 2, _NCHUNK - 1):
        wait_scatter(c % _NB)

    plsc.subcore_barrier()
    _dump_accum(accum, se_hbm, sid, cid)


_SC_CP = pltpu.CompilerParams()
if "needs_layout_passes" in pltpu.CompilerParams.__dataclass_fields__:
    _SC_CP = dataclasses.replace(_SC_CP, needs_layout_passes=False)

_SC_MESH = plsc.VectorSubcoreMesh(core_axis_name="c", subcore_axis_name="s")


@functools.partial(
    pl.kernel,
    compiler_params=_SC_CP,
    out_type=[
        jax.ShapeDtypeStruct((E, D), jnp.float32),
        jax.ShapeDtypeStruct((2 * N, D), jnp.float32),
    ],
    mesh=_SC_MESH,
    scratch_types=[
        pltpu.VMEM((_NB, _C), jnp.int32),
        pltpu.VMEM((_NB, _C), jnp.int32),
        pltpu.VMEM((_NB, _C, D), jnp.float32),
        pltpu.VMEM((_NB, _C, D), jnp.float32),
        pltpu.VMEM_SHARED((N, D), jnp.float32),
        pltpu.SemaphoreType.DMA((_NB,)),
        pltpu.SemaphoreType.DMA((_NB,)),
        pltpu.SemaphoreType.DMA((_NB,)),
        pltpu.SemaphoreType.DMA((_NB,)),
    ],
)
def _sc_gather(*args):
    _sc_gather_body(*args)


@functools.partial(
    pl.kernel,
    compiler_params=_SC_CP,
    out_type=jax.ShapeDtypeStruct((2 * N, D), jnp.float32),
    mesh=_SC_MESH,
    scratch_types=[
        pltpu.VMEM((_NB, _C), jnp.int32),
        pltpu.VMEM((_NB, _C, D), jnp.float32),
        pltpu.VMEM_SHARED((N, D), jnp.float32),
        pltpu.SemaphoreType.DMA((_NB,)),
        pltpu.SemaphoreType.DMA((_NB,)),
    ],
)
def _sc_segsum(*args):
    _sc_segsum_body(*args)


def kernel(nodes, edges, senders, receivers, W_e, b_e, W_n, b_n,
           gamma_n, beta_n, gamma_e, beta_e):
    W1, W2, W3 = W_e[:D], W_e[D:2 * D], W_e[2 * D:]
    b_e2 = b_e.reshape(1, D)
    g_e2, bt_e2 = gamma_e.reshape(1, D), beta_e.reshape(1, D)
    g_n2, bt_n2 = gamma_n.reshape(1, D), beta_n.reshape(1, D)

    p1, p2 = pl.pallas_call(
        _proj_body,
        out_shape=[jax.ShapeDtypeStruct((N, D), jnp.float32)] * 2,
    )(nodes, W2, W3, b_e2)

    gp, rg2 = _sc_gather(p1, p2, senders, receivers)
    se2 = _sc_segsum(edges, receivers)

    BE = 4000
    edges_out = pl.pallas_call(
        _edge_ep_body,
        grid=(E // BE,),
        in_specs=[
            pl.BlockSpec((BE, D), lambda i: (i, 0)),
            pl.BlockSpec((BE, D), lambda i: (i, 0)),
            pl.BlockSpec((D, D), lambda i: (0, 0)),
            pl.BlockSpec((1, D), lambda i: (0, 0)),
            pl.BlockSpec((1, D), lambda i: (0, 0)),
        ],
        out_specs=pl.BlockSpec((BE, D), lambda i: (i, 0)),
        out_shape=jax.ShapeDtypeStruct((E, D), jnp.float32),
    )(edges, gp, W1, g_e2, bt_e2)

    nodes_out = pl.pallas_call(
        _node_body,
        out_shape=jax.ShapeDtypeStruct((N, D), jnp.float32),
    )(nodes, se2, rg2, W1, W_n[:D], W_n[D:], b_n.reshape(1, D), g_n2, bt_n2)

    return nodes_out, edges_out


# speedup vs baseline: 5.4314x; 1.0001x over previous
import dataclasses
import functools

import jax
import jax.numpy as jnp
from jax import lax
from jax.experimental import pallas as pl
from jax.experimental.pallas import tpu as pltpu
from jax.experimental.pallas import tpu_sc as plsc

N = 10000
E = 320000
D = 128

_NC = 2
_NS = 16
_NW = _NC * _NS
_EPW = E // _NW
_C = 16
_NCHUNK = _EPW // _C
_NB = 5
_NPW = N // _NS


def _proj_body(nodes_ref, w2_ref, w3_ref, be_ref, p1_ref, p2_ref):
    n = nodes_ref[...]
    p1_ref[...] = jnp.dot(
        n, w2_ref[...], preferred_element_type=jnp.float32
    )
    p2_ref[...] = (
        jnp.dot(n, w3_ref[...], preferred_element_type=jnp.float32)
        + be_ref[...]
    )


def _ln(y, g, b):
    mean = jnp.mean(y, axis=-1, keepdims=True)
    yc = y - mean
    var = jnp.mean(yc * yc, axis=-1, keepdims=True)
    return yc * lax.rsqrt(var + 1e-6) * g + b


def _edge_ep_body(e_ref, gp_ref, w1_ref, g_ref, b_ref, out_ref):
    e = e_ref[...]
    ne = (
        jnp.dot(e, w1_ref[...], preferred_element_type=jnp.float32)
        + gp_ref[...]
    )
    out_ref[...] = _ln(ne + e, g_ref[...], b_ref[...])


def _node_body(nodes_ref, se_ref, rg_ref, w1_ref, wn1_ref, wn2_ref, bn_ref,
               g_ref, b_ref, out_ref):
    nodes = nodes_ref[...]
    se = se_ref[:N, :] + se_ref[N:, :]
    r = (
        jnp.dot(se, w1_ref[...], preferred_element_type=jnp.float32)
        + rg_ref[:N, :] + rg_ref[N:, :]
    )
    nn = (
        jnp.dot(nodes, wn1_ref[...], preferred_element_type=jnp.float32)
        + jnp.dot(r, wn2_ref[...], preferred_element_type=jnp.float32)
        + bn_ref[...]
    )
    out_ref[...] = _ln(nn + nodes, g_ref[...], b_ref[...])


def _zero_accum(p1_v, accum, row0):
    def zfill(i, _):
        for g in range(D // 16):
            p1_v[0, i, pl.ds(g * 16, 16)] = jnp.zeros((16,), jnp.float32)
        return 0
    lax.fori_loop(0, _C, zfill, 0)

    def zcopy(k, _):
        pltpu.sync_copy(p1_v.at[0], accum.at[pl.ds(row0 + k * _C, _C)])
        return 0
    lax.fori_loop(0, _NPW // _C, zcopy, 0)
    pltpu.sync_copy(p1_v.at[0, pl.ds(0, 1)],
                    accum.at[pl.ds(row0 + (_NPW // _C) * _C, 1)])


def _dump_accum(accum, dst_hbm, sid, cid):
    o = sid * 624

    @pl.when(sid < _NS - 1)
    def _():
        pltpu.sync_copy(accum.at[pl.ds(o, 624)],
                        dst_hbm.at[pl.ds(cid * N + o, 624)])

    @pl.when(sid == _NS - 1)
    def _():
        pltpu.sync_copy(accum.at[pl.ds(o, 640)],
                        dst_hbm.at[pl.ds(cid * N + o, 640)])


def _sc_gather_body(p1_hbm, p2_hbm, s_hbm, r_hbm,
                    g_hbm, rg_hbm,
                    s_v, r_v, p1_v, p2_v, accum,
                    lsem, gsem, wsem, asem):
    cid = lax.axis_index("c")
    sid = lax.axis_index("s")
    wid = sid * _NC + cid
    row0 = sid * _NPW

    _zero_accum(p1_v, accum, row0)
    plsc.subcore_barrier()

    def _base(c):
        return wid * _EPW + c * _C

    def issue_loads(c, j):
        b = _base(c)
        pltpu.async_copy(s_hbm.at[pl.ds(b, _C)], s_v.at[j], lsem.at[j])
        pltpu.async_copy(r_hbm.at[pl.ds(b, _C)], r_v.at[j], lsem.at[j])

    def wait_loads(c, j):
        b = _base(c)
        pltpu.make_async_copy(s_hbm.at[pl.ds(b, _C)], s_v.at[j],
                              lsem.at[j]).wait()
        pltpu.make_async_copy(r_hbm.at[pl.ds(b, _C)], r_v.at[j],
                              lsem.at[j]).wait()

    def issue_gathers(j):
        pltpu.async_copy(p1_hbm.at[s_v.at[j]], p1_v.at[j], gsem.at[j])
        pltpu.async_copy(p2_hbm.at[r_v.at[j]], p2_v.at[j], gsem.at[j])

    def wait_gathers(j):
        pltpu.make_async_copy(p1_hbm.at[s_v.at[j]], p1_v.at[j],
                              gsem.at[j]).wait()
        pltpu.make_async_copy(p2_hbm.at[r_v.at[j]], p2_v.at[j],
                              gsem.at[j]).wait()

    def issue_writes(c, j):
        b = _base(c)
        pltpu.async_copy(p1_v.at[j], g_hbm.at[pl.ds(b, _C)], wsem.at[j])
        pltpu.async_copy(p1_v.at[j], accum.at[r_v.at[j]], asem.at[j],
                         add=True)

    def wait_writes(c, j):
        b = _base(c)
        pltpu.make_async_copy(p1_v.at[j], g_hbm.at[pl.ds(b, _C)],
                              wsem.at[j]).wait()
        pltpu.make_async_copy(p1_v.at[j], accum.at[r_v.at[j]],
                              asem.at[j]).wait()

    issue_loads(0, 0)
    issue_loads(1, 1)
    issue_loads(2, 2)
    wait_loads(0, 0)
    issue_gathers(0)
    wait_loads(1, 1)
    issue_gathers(1)

    def outer(k, _):
        for jj in range(_NB):
            c = k * _NB + jj
            j = jj
            jg = (jj + 2) % _NB
            jl = (jj + 3) % _NB

            @pl.when(c >= 2)
            def _():
                wait_writes(c - 2, jl)

            @pl.when(c + 3 < _NCHUNK)
            def _():
                issue_loads(c + 3, jl)

            @pl.when(c + 2 < _NCHUNK)
            def _():
                wait_loads(c + 2, jg)
                issue_gathers(jg)

            wait_gathers(j)

            p1s, p2s = p1_v.at[j], p2_v.at[j]

            def row(i, _):
                for g in range(D // 16):
                    sl = pl.ds(g * 16, 16)
                    p1s[i, sl] = p1s[i, sl] + p2s[i, sl]
                return 0
            lax.fori_loop(0, _C, row, 0)

            issue_writes(c, j)
        return 0
    lax.fori_loop(0, _NCHUNK // _NB, outer, 0)
    for c in (_NCHUNK - 2, _NCHUNK - 1):
        wait_writes(c, c % _NB)

    plsc.subcore_barrier()
    _dump_accum(accum, rg_hbm, sid, cid)


def _sc_segsum_body(edges_hbm, r_hbm, tok_hbm,
                    se_hbm,
                    r_v, p1_v, accum, lsem, asem):
    cid = lax.axis_index("c")
    sid = lax.axis_index("s")
    wid = sid * _NC + cid
    row0 = sid * _NPW

    _zero_accum(p1_v, accum, row0)
    plsc.subcore_barrier()

    def _base(c):
        return wid * _EPW + c * _C

    def issue_loads(c, j):
        b = _base(c)
        pltpu.async_copy(r_hbm.at[pl.ds(b, _C)], r_v.at[j], lsem.at[j])
        pltpu.async_copy(edges_hbm.at[pl.ds(b, _C)], p1_v.at[j], lsem.at[j])

    def wait_loads(c, j):
        b = _base(c)
        pltpu.make_async_copy(r_hbm.at[pl.ds(b, _C)], r_v.at[j],
                              lsem.at[j]).wait()
        pltpu.make_async_copy(edges_hbm.at[pl.ds(b, _C)], p1_v.at[j],
                              lsem.at[j]).wait()

    def issue_scatter(j):
        pltpu.async_copy(p1_v.at[j], accum.at[r_v.at[j]], asem.at[j],
                         add=True)

    def wait_scatter(j):
        pltpu.make_async_copy(p1_v.at[j], accum.at[r_v.at[j]],
                              asem.at[j]).wait()

    issue_loads(0, 0)
    issue_loads(1, 1)

    def outer(k, _):
        for jj in range(_NB):
            c = k * _NB + jj
            j = jj
            jl = (jj + 2) % _NB

            @pl.when(c >= 3)
            def _():
                wait_scatter(jl)

            @pl.when(c + 2 < _NCHUNK)
            def _():
                issue_loads(c + 2, jl)

            wait_loads(c, j)
            issue_scatter(j)
        return 0
    lax.fori_loop(0, _NCHUNK // _NB, outer, 0)
    for c in (_NCHUNK - 3, _NCHUNK - 2, _NCHUNK - 1):
        wait_scatter(c % _NB)

    plsc.subcore_barrier()
    _dump_accum(accum, se_hbm, sid, cid)


_SC_CP = pltpu.CompilerParams()
if "needs_layout_passes" in pltpu.CompilerParams.__dataclass_fields__:
    _SC_CP = dataclasses.replace(_SC_CP, needs_layout_passes=False)

_SC_MESH = plsc.VectorSubcoreMesh(core_axis_name="c", subcore_axis_name="s")


@functools.partial(
    pl.kernel,
    compiler_params=_SC_CP,
    out_type=[
        jax.ShapeDtypeStruct((E, D), jnp.float32),
        jax.ShapeDtypeStruct((2 * N, D), jnp.float32),
    ],
    mesh=_SC_MESH,
    scratch_types=[
        pltpu.VMEM((_NB, _C), jnp.int32),
        pltpu.VMEM((_NB, _C), jnp.int32),
        pltpu.VMEM((_NB, _C, D), jnp.float32),
        pltpu.VMEM((_NB, _C, D), jnp.float32),
        pltpu.VMEM_SHARED((N, D), jnp.float32),
        pltpu.SemaphoreType.DMA((_NB,)),
        pltpu.SemaphoreType.DMA((_NB,)),
        pltpu.SemaphoreType.DMA((_NB,)),
        pltpu.SemaphoreType.DMA((_NB,)),
    ],
)
def _sc_gather(*args):
    _sc_gather_body(*args)


@functools.partial(
    pl.kernel,
    compiler_params=_SC_CP,
    out_type=jax.ShapeDtypeStruct((2 * N, D), jnp.float32),
    mesh=_SC_MESH,
    scratch_types=[
        pltpu.VMEM((_NB, _C), jnp.int32),
        pltpu.VMEM((_NB, _C, D), jnp.float32),
        pltpu.VMEM_SHARED((N, D), jnp.float32),
        pltpu.SemaphoreType.DMA((_NB,)),
        pltpu.SemaphoreType.DMA((_NB,)),
    ],
)
def _sc_segsum(*args):
    _sc_segsum_body(*args)


def kernel(nodes, edges, senders, receivers, W_e, b_e, W_n, b_n,
           gamma_n, beta_n, gamma_e, beta_e):
    W1, W2, W3 = W_e[:D], W_e[D:2 * D], W_e[2 * D:]
    b_e2 = b_e.reshape(1, D)
    g_e2, bt_e2 = gamma_e.reshape(1, D), beta_e.reshape(1, D)
    g_n2, bt_n2 = gamma_n.reshape(1, D), beta_n.reshape(1, D)

    p1, p2 = pl.pallas_call(
        _proj_body,
        out_shape=[jax.ShapeDtypeStruct((N, D), jnp.float32)] * 2,
    )(nodes, W2, W3, b_e2)

    gp, rg2 = _sc_gather(p1, p2, senders, receivers)
    se2 = _sc_segsum(edges, receivers, rg2)

    BE = 4000
    edges_out = pl.pallas_call(
        _edge_ep_body,
        grid=(E // BE,),
        in_specs=[
            pl.BlockSpec((BE, D), lambda i: (i, 0)),
            pl.BlockSpec((BE, D), lambda i: (i, 0)),
            pl.BlockSpec((D, D), lambda i: (0, 0)),
            pl.BlockSpec((1, D), lambda i: (0, 0)),
            pl.BlockSpec((1, D), lambda i: (0, 0)),
        ],
        out_specs=pl.BlockSpec((BE, D), lambda i: (i, 0)),
        out_shape=jax.ShapeDtypeStruct((E, D), jnp.float32),
    )(edges, gp, W1, g_e2, bt_e2)

    nodes_out = pl.pallas_call(
        _node_body,
        out_shape=jax.ShapeDtypeStruct((N, D), jnp.float32),
    )(nodes, se2, rg2, W1, W_n[:D], W_n[D:], b_n.reshape(1, D), g_n2, bt_n2)

    return nodes_out, edges_out


# speedup vs baseline: 5.7353x; 1.0559x over previous
import dataclasses
import functools

import jax
import jax.numpy as jnp
from jax import lax
from jax.experimental import pallas as pl
from jax.experimental.pallas import tpu as pltpu
from jax.experimental.pallas import tpu_sc as plsc

N = 10000
E = 320000
D = 128

_NC = 2
_NS = 16
_NW = _NC * _NS
_EPW = E // _NW
_C = 16
_NCHUNK = _EPW // _C
_NB = 5
_NPW = N // _NS
_C1 = 40
_NCHUNK1 = _EPW // _C1


def _proj_body(nodes_ref, w2_ref, w3_ref, be_ref, p1_ref, p2_ref):
    n = nodes_ref[...]
    p1_ref[...] = jnp.dot(
        n, w2_ref[...], preferred_element_type=jnp.float32
    )
    p2_ref[...] = (
        jnp.dot(n, w3_ref[...], preferred_element_type=jnp.float32)
        + be_ref[...]
    )


def _ln(y, g, b):
    mean = jnp.mean(y, axis=-1, keepdims=True)
    yc = y - mean
    var = jnp.mean(yc * yc, axis=-1, keepdims=True)
    return yc * lax.rsqrt(var + 1e-6) * g + b


def _edge_ep_body(e_ref, gp_ref, w1_ref, g_ref, b_ref, out_ref):
    e = e_ref[...]
    ne = (
        jnp.dot(e, w1_ref[...], preferred_element_type=jnp.float32)
        + gp_ref[...]
    )
    out_ref[...] = _ln(ne + e, g_ref[...], b_ref[...])


def _node_body(nodes_ref, se_ref, rg_ref, w1_ref, wn1_ref, wn2_ref, bn_ref,
               g_ref, b_ref, out_ref):
    nodes = nodes_ref[...]
    se = se_ref[:N, :] + se_ref[N:, :]
    r = (
        jnp.dot(se, w1_ref[...], preferred_element_type=jnp.float32)
        + rg_ref[:N, :] + rg_ref[N:, :]
    )
    nn = (
        jnp.dot(nodes, wn1_ref[...], preferred_element_type=jnp.float32)
        + jnp.dot(r, wn2_ref[...], preferred_element_type=jnp.float32)
        + bn_ref[...]
    )
    out_ref[...] = _ln(nn + nodes, g_ref[...], b_ref[...])


def _zero_accum(p1_v, accum, row0):
    def zfill(i, _):
        for g in range(D // 16):
            p1_v[0, i, pl.ds(g * 16, 16)] = jnp.zeros((16,), jnp.float32)
        return 0
    lax.fori_loop(0, _C, zfill, 0)

    def zcopy(k, _):
        pltpu.sync_copy(p1_v.at[0], accum.at[pl.ds(row0 + k * _C, _C)])
        return 0
    lax.fori_loop(0, _NPW // _C, zcopy, 0)
    pltpu.sync_copy(p1_v.at[0, pl.ds(0, 1)],
                    accum.at[pl.ds(row0 + (_NPW // _C) * _C, 1)])


def _dump_accum(accum, dst_hbm, sid, cid):
    o = sid * 624

    @pl.when(sid < _NS - 1)
    def _():
        pltpu.sync_copy(accum.at[pl.ds(o, 624)],
                        dst_hbm.at[pl.ds(cid * N + o, 624)])

    @pl.when(sid == _NS - 1)
    def _():
        pltpu.sync_copy(accum.at[pl.ds(o, 640)],
                        dst_hbm.at[pl.ds(cid * N + o, 640)])


def _sc_gather_body(p1_hbm, p2_hbm, s_hbm, r_hbm,
                    g_hbm, rg_hbm,
                    s_v, r_v, p1_v, p2_v, accum,
                    lsem, gsem, wsem, asem):
    cid = lax.axis_index("c")
    sid = lax.axis_index("s")
    wid = sid * _NC + cid
    row0 = sid * _NPW

    _zero_accum(p1_v, accum, row0)
    plsc.subcore_barrier()

    def _base(c):
        return wid * _EPW + c * _C

    def issue_loads(c, j):
        b = _base(c)
        pltpu.async_copy(s_hbm.at[pl.ds(b, _C)], s_v.at[j], lsem.at[j])
        pltpu.async_copy(r_hbm.at[pl.ds(b, _C)], r_v.at[j], lsem.at[j])

    def wait_loads(c, j):
        b = _base(c)
        pltpu.make_async_copy(s_hbm.at[pl.ds(b, _C)], s_v.at[j],
                              lsem.at[j]).wait()
        pltpu.make_async_copy(r_hbm.at[pl.ds(b, _C)], r_v.at[j],
                              lsem.at[j]).wait()

    def issue_gathers(j):
        pltpu.async_copy(p1_hbm.at[s_v.at[j]], p1_v.at[j], gsem.at[j])
        pltpu.async_copy(p2_hbm.at[r_v.at[j]], p2_v.at[j], gsem.at[j])

    def wait_gathers(j):
        pltpu.make_async_copy(p1_hbm.at[s_v.at[j]], p1_v.at[j],
                              gsem.at[j]).wait()
        pltpu.make_async_copy(p2_hbm.at[r_v.at[j]], p2_v.at[j],
                              gsem.at[j]).wait()

    def issue_writes(c, j):
        b = _base(c)
        pltpu.async_copy(p1_v.at[j], g_hbm.at[pl.ds(b, _C)], wsem.at[j])
        pltpu.async_copy(p1_v.at[j], accum.at[r_v.at[j]], asem.at[j],
                         add=True)

    def wait_writes(c, j):
        b = _base(c)
        pltpu.make_async_copy(p1_v.at[j], g_hbm.at[pl.ds(b, _C)],
                              wsem.at[j]).wait()
        pltpu.make_async_copy(p1_v.at[j], accum.at[r_v.at[j]],
                              asem.at[j]).wait()

    issue_loads(0, 0)
    issue_loads(1, 1)
    issue_loads(2, 2)
    wait_loads(0, 0)
    issue_gathers(0)
    wait_loads(1, 1)
    issue_gathers(1)

    def outer(k, _):
        for jj in range(_NB):
            c = k * _NB + jj
            j = jj
            jg = (jj + 2) % _NB
            jl = (jj + 3) % _NB

            @pl.when(c >= 2)
            def _():
                wait_writes(c - 2, jl)

            @pl.when(c + 3 < _NCHUNK)
            def _():
                issue_loads(c + 3, jl)

            @pl.when(c + 2 < _NCHUNK)
            def _():
                wait_loads(c + 2, jg)
                issue_gathers(jg)

            wait_gathers(j)

            p1s, p2s = p1_v.at[j], p2_v.at[j]

            def row(i, _):
                for g in range(D // 16):
                    sl = pl.ds(g * 16, 16)
                    p1s[i, sl] = p1s[i, sl] + p2s[i, sl]
                return 0
            lax.fori_loop(0, _C, row, 0)

            issue_writes(c, j)
        return 0
    lax.fori_loop(0, _NCHUNK // _NB, outer, 0)
    for c in (_NCHUNK - 2, _NCHUNK - 1):
        wait_writes(c, c % _NB)

    plsc.subcore_barrier()
    _dump_accum(accum, rg_hbm, sid, cid)


def _sc_segsum_body(edges_hbm, r_hbm, tok_hbm,
                    se_hbm,
                    r_v, p1_v, accum, lsem, asem):
    cid = lax.axis_index("c")
    sid = lax.axis_index("s")
    wid = sid * _NC + cid
    row0 = sid * _NPW

    def zfill(i, _):
        for g in range(D // 16):
            p1_v[0, i, pl.ds(g * 16, 16)] = jnp.zeros((16,), jnp.float32)
        return 0
    lax.fori_loop(0, _C1, zfill, 0)

    def zcopy(k, _):
        pltpu.sync_copy(p1_v.at[0], accum.at[pl.ds(row0 + k * _C1, _C1)])
        return 0
    lax.fori_loop(0, _NPW // _C1, zcopy, 0)
    pltpu.sync_copy(p1_v.at[0, pl.ds(0, _NPW - (_NPW // _C1) * _C1)],
                    accum.at[pl.ds(row0 + (_NPW // _C1) * _C1,
                                   _NPW - (_NPW // _C1) * _C1)])
    plsc.subcore_barrier()

    def _base(c):
        return wid * _EPW + c * _C1

    def issue_loads(c, j):
        b = _base(c)
        pltpu.async_copy(r_hbm.at[pl.ds(b, _C1)], r_v.at[j], lsem.at[j])
        pltpu.async_copy(edges_hbm.at[pl.ds(b, _C1)], p1_v.at[j], lsem.at[j])

    def wait_loads(c, j):
        b = _base(c)
        pltpu.make_async_copy(r_hbm.at[pl.ds(b, _C1)], r_v.at[j],
                              lsem.at[j]).wait()
        pltpu.make_async_copy(edges_hbm.at[pl.ds(b, _C1)], p1_v.at[j],
                              lsem.at[j]).wait()

    def issue_scatter(j):
        pltpu.async_copy(p1_v.at[j], accum.at[r_v.at[j]], asem.at[j],
                         add=True)

    def wait_scatter(j):
        pltpu.make_async_copy(p1_v.at[j], accum.at[r_v.at[j]],
                              asem.at[j]).wait()

    issue_loads(0, 0)
    issue_loads(1, 1)

    def outer(k, _):
        for jj in range(_NB):
            c = k * _NB + jj
            j = jj
            jl = (jj + 2) % _NB

            @pl.when(c >= 3)
            def _():
                wait_scatter(jl)

            @pl.when(c + 2 < _NCHUNK1)
            def _():
                issue_loads(c + 2, jl)

            wait_loads(c, j)
            issue_scatter(j)
        return 0
    lax.fori_loop(0, _NCHUNK1 // _NB, outer, 0)
    for c in (_NCHUNK1 - 3, _NCHUNK1 - 2, _NCHUNK1 - 1):
        wait_scatter(c % _NB)

    plsc.subcore_barrier()
    _dump_accum(accum, se_hbm, sid, cid)


_SC_CP = pltpu.CompilerParams()
if "needs_layout_passes" in pltpu.CompilerParams.__dataclass_fields__:
    _SC_CP = dataclasses.replace(_SC_CP, needs_layout_passes=False)

_SC_MESH = plsc.VectorSubcoreMesh(core_axis_name="c", subcore_axis_name="s")


@functools.partial(
    pl.kernel,
    compiler_params=_SC_CP,
    out_type=[
        jax.ShapeDtypeStruct((E, D), jnp.float32),
        jax.ShapeDtypeStruct((2 * N, D), jnp.float32),
    ],
    mesh=_SC_MESH,
    scratch_types=[
        pltpu.VMEM((_NB, _C), jnp.int32),
        pltpu.VMEM((_NB, _C), jnp.int32),
        pltpu.VMEM((_NB, _C, D), jnp.float32),
        pltpu.VMEM((_NB, _C, D), jnp.float32),
        pltpu.VMEM_SHARED((N, D), jnp.float32),
        pltpu.SemaphoreType.DMA((_NB,)),
        pltpu.SemaphoreType.DMA((_NB,)),
        pltpu.SemaphoreType.DMA((_NB,)),
        pltpu.SemaphoreType.DMA((_NB,)),
    ],
)
def _sc_gather(*args):
    _sc_gather_body(*args)


@functools.partial(
    pl.kernel,
    compiler_params=_SC_CP,
    out_type=jax.ShapeDtypeStruct((2 * N, D), jnp.float32),
    mesh=_SC_MESH,
    scratch_types=[
        pltpu.VMEM((_NB, _C1), jnp.int32),
        pltpu.VMEM((_NB, _C1, D), jnp.float32),
        pltpu.VMEM_SHARED((N, D), jnp.float32),
        pltpu.SemaphoreType.DMA((_NB,)),
        pltpu.SemaphoreType.DMA((_NB,)),
    ],
)
def _sc_segsum(*args):
    _sc_segsum_body(*args)


def kernel(nodes, edges, senders, receivers, W_e, b_e, W_n, b_n,
           gamma_n, beta_n, gamma_e, beta_e):
    W1, W2, W3 = W_e[:D], W_e[D:2 * D], W_e[2 * D:]
    b_e2 = b_e.reshape(1, D)
    g_e2, bt_e2 = gamma_e.reshape(1, D), beta_e.reshape(1, D)
    g_n2, bt_n2 = gamma_n.reshape(1, D), beta_n.reshape(1, D)

    p1, p2 = pl.pallas_call(
        _proj_body,
        out_shape=[jax.ShapeDtypeStruct((N, D), jnp.float32)] * 2,
    )(nodes, W2, W3, b_e2)

    gp, rg2 = _sc_gather(p1, p2, senders, receivers)
    se2 = _sc_segsum(edges, receivers, rg2)

    BE = 4000
    edges_out = pl.pallas_call(
        _edge_ep_body,
        grid=(E // BE,),
        in_specs=[
            pl.BlockSpec((BE, D), lambda i: (i, 0)),
            pl.BlockSpec((BE, D), lambda i: (i, 0)),
            pl.BlockSpec((D, D), lambda i: (0, 0)),
            pl.BlockSpec((1, D), lambda i: (0, 0)),
            pl.BlockSpec((1, D), lambda i: (0, 0)),
        ],
        out_specs=pl.BlockSpec((BE, D), lambda i: (i, 0)),
        out_shape=jax.ShapeDtypeStruct((E, D), jnp.float32),
    )(edges, gp, W1, g_e2, bt_e2)

    nodes_out = pl.pallas_call(
        _node_body,
        out_shape=jax.ShapeDtypeStruct((N, D), jnp.float32),
    )(nodes, se2, rg2, W1, W_n[:D], W_n[D:], b_n.reshape(1, D), g_n2, bt_n2)

    return nodes_out, edges_out


# speedup vs baseline: 6.9241x; 1.2073x over previous
import dataclasses
import functools

import jax
import jax.numpy as jnp
from jax import lax
from jax.experimental import pallas as pl
from jax.experimental.pallas import tpu as pltpu
from jax.experimental.pallas import tpu_sc as plsc

N = 10000
E = 320000
D = 128

_NC = 2
_NS = 16
_NW = _NC * _NS
_EPW = E // _NW
_C = 16
_NCHUNK = _EPW // _C
_NB = 5
_NPW = N // _NS
_C1 = 40
_NCHUNK1 = _EPW // _C1
_C2 = 40
_NCHUNK2 = _EPW // _C2
_NBD = 4
_NBI = 8


def _proj_body(nodes_ref, w2_ref, w3_ref, be_ref, p1_ref, p2_ref):
    n = nodes_ref[...]
    p1_ref[...] = jnp.dot(
        n, w2_ref[...], preferred_element_type=jnp.float32
    )
    p2_ref[...] = (
        jnp.dot(n, w3_ref[...], preferred_element_type=jnp.float32)
        + be_ref[...]
    )


def _ln(y, g, b):
    mean = jnp.mean(y, axis=-1, keepdims=True)
    yc = y - mean
    var = jnp.mean(yc * yc, axis=-1, keepdims=True)
    return yc * lax.rsqrt(var + 1e-6) * g + b


def _edge_ep_body(e_ref, gp_ref, w1_ref, g_ref, b_ref, out_ref):
    e = e_ref[...]
    ne = (
        jnp.dot(e, w1_ref[...], preferred_element_type=jnp.float32)
        + gp_ref[...]
    )
    out_ref[...] = _ln(ne + e, g_ref[...], b_ref[...])


def _node_body(nodes_ref, se_ref, rg_ref, w1_ref, wn1_ref, wn2_ref, bn_ref,
               g_ref, b_ref, out_ref):
    nodes = nodes_ref[...]
    se = se_ref[:N, :] + se_ref[N:, :]
    r = (
        jnp.dot(se, w1_ref[...], preferred_element_type=jnp.float32)
        + rg_ref[:N, :] + rg_ref[N:, :]
    )
    nn = (
        jnp.dot(nodes, wn1_ref[...], preferred_element_type=jnp.float32)
        + jnp.dot(r, wn2_ref[...], preferred_element_type=jnp.float32)
        + bn_ref[...]
    )
    out_ref[...] = _ln(nn + nodes, g_ref[...], b_ref[...])


def _zero_accum(p1_v, accum, row0):
    def zfill(i, _):
        for g in range(D // 16):
            p1_v[0, i, pl.ds(g * 16, 16)] = jnp.zeros((16,), jnp.float32)
        return 0
    lax.fori_loop(0, _C, zfill, 0)

    def zcopy(k, _):
        pltpu.sync_copy(p1_v.at[0], accum.at[pl.ds(row0 + k * _C, _C)])
        return 0
    lax.fori_loop(0, _NPW // _C, zcopy, 0)
    pltpu.sync_copy(p1_v.at[0, pl.ds(0, 1)],
                    accum.at[pl.ds(row0 + (_NPW // _C) * _C, 1)])


def _dump_accum(accum, dst_hbm, sid, cid):
    o = sid * 624

    @pl.when(sid < _NS - 1)
    def _():
        pltpu.sync_copy(accum.at[pl.ds(o, 624)],
                        dst_hbm.at[pl.ds(cid * N + o, 624)])

    @pl.when(sid == _NS - 1)
    def _():
        pltpu.sync_copy(accum.at[pl.ds(o, 640)],
                        dst_hbm.at[pl.ds(cid * N + o, 640)])


def _sc_gather_body(p1_hbm, p2_hbm, s_hbm, r_hbm,
                    g_hbm, rg_hbm,
                    s_v, r_v, p1_v, p2_v, accum,
                    lsem, gsem, wsem, asem):
    cid = lax.axis_index("c")
    sid = lax.axis_index("s")
    wid = sid * _NC + cid
    row0 = sid * _NPW

    def zfill(i, _):
        for g in range(D // 16):
            p1_v[0, i, pl.ds(g * 16, 16)] = jnp.zeros((16,), jnp.float32)
        return 0
    lax.fori_loop(0, _C2, zfill, 0)

    def zcopy(k, _):
        pltpu.sync_copy(p1_v.at[0], accum.at[pl.ds(row0 + k * _C2, _C2)])
        return 0
    lax.fori_loop(0, _NPW // _C2, zcopy, 0)
    pltpu.sync_copy(p1_v.at[0, pl.ds(0, _NPW - (_NPW // _C2) * _C2)],
                    accum.at[pl.ds(row0 + (_NPW // _C2) * _C2,
                                   _NPW - (_NPW // _C2) * _C2)])
    plsc.subcore_barrier()

    def _base(c):
        return wid * _EPW + c * _C2

    def issue_loads(c, j):
        b = _base(c)
        pltpu.async_copy(s_hbm.at[pl.ds(b, _C2)], s_v.at[j], lsem.at[j])
        pltpu.async_copy(r_hbm.at[pl.ds(b, _C2)], r_v.at[j], lsem.at[j])

    def wait_loads(c, j):
        b = _base(c)
        pltpu.make_async_copy(s_hbm.at[pl.ds(b, _C2)], s_v.at[j],
                              lsem.at[j]).wait()
        pltpu.make_async_copy(r_hbm.at[pl.ds(b, _C2)], r_v.at[j],
                              lsem.at[j]).wait()

    def issue_gathers(ji, jd):
        pltpu.async_copy(p1_hbm.at[s_v.at[ji]], p1_v.at[jd], gsem.at[jd])
        pltpu.async_copy(p2_hbm.at[r_v.at[ji]], p2_v.at[jd], gsem.at[jd])

    def wait_gathers(ji, jd):
        pltpu.make_async_copy(p1_hbm.at[s_v.at[ji]], p1_v.at[jd],
                              gsem.at[jd]).wait()
        pltpu.make_async_copy(p2_hbm.at[r_v.at[ji]], p2_v.at[jd],
                              gsem.at[jd]).wait()

    def issue_writes(c, ji, jd):
        b = _base(c)
        pltpu.async_copy(p1_v.at[jd], g_hbm.at[pl.ds(b, _C2)], wsem.at[jd])
        pltpu.async_copy(p1_v.at[jd], accum.at[r_v.at[ji]], asem.at[jd],
                         add=True)

    def wait_writes(c, ji, jd):
        b = _base(c)
        pltpu.make_async_copy(p1_v.at[jd], g_hbm.at[pl.ds(b, _C2)],
                              wsem.at[jd]).wait()
        pltpu.make_async_copy(p1_v.at[jd], accum.at[r_v.at[ji]],
                              asem.at[jd]).wait()

    def compute(jd):
        p1s, p2s = p1_v.at[jd], p2_v.at[jd]

        def row(i, _):
            for g in range(D // 16):
                sl = pl.ds(g * 16, 16)
                p1s[i, sl] = p1s[i, sl] + p2s[i, sl]
            return 0
        lax.fori_loop(0, _C2, row, 0)

    for c in range(4):
        issue_loads(c, c)
    wait_loads(0, 0)
    issue_gathers(0, 0)
    wait_loads(1, 1)
    issue_gathers(1, 1)

    _MAIN = _NCHUNK2 - (_NCHUNK2 % _NBI)

    def outer(k, _):
        for jj in range(_NBI):
            c = k * _NBI + jj
            ji = jj
            jd = jj % _NBD
            jdg = (jj + 2) % _NBD
            jig = (jj + 2) % _NBI
            jiw = (jj - 2) % _NBI
            jil = (jj + 4) % _NBI

            @pl.when(c >= 2)
            def _():
                wait_writes(c - 2, jiw, jdg)

            @pl.when(c + 4 < _NCHUNK2)
            def _():
                issue_loads(c + 4, jil)

            @pl.when(c + 2 < _NCHUNK2)
            def _():
                wait_loads(c + 2, jig)
                issue_gathers(jig, jdg)

            wait_gathers(ji, jd)
            compute(jd)
            issue_writes(c, ji, jd)
        return 0
    lax.fori_loop(0, _MAIN // _NBI, outer, 0)

    for c in range(_MAIN, _NCHUNK2):
        wait_writes(c - 2, (c - 2) % _NBI, (c - 2) % _NBD)
        wait_gathers(c % _NBI, c % _NBD)
        compute(c % _NBD)
        issue_writes(c, c % _NBI, c % _NBD)
    for c in (_NCHUNK2 - 2, _NCHUNK2 - 1):
        wait_writes(c, c % _NBI, c % _NBD)

    plsc.subcore_barrier()
    _dump_accum(accum, rg_hbm, sid, cid)


def _sc_segsum_body(edges_hbm, r_hbm, tok_hbm,
                    se_hbm,
                    r_v, p1_v, accum, lsem, asem):
    cid = lax.axis_index("c")
    sid = lax.axis_index("s")
    wid = sid * _NC + cid
    row0 = sid * _NPW

    def zfill(i, _):
        for g in range(D // 16):
            p1_v[0, i, pl.ds(g * 16, 16)] = jnp.zeros((16,), jnp.float32)
        return 0
    lax.fori_loop(0, _C1, zfill, 0)

    def zcopy(k, _):
        pltpu.sync_copy(p1_v.at[0], accum.at[pl.ds(row0 + k * _C1, _C1)])
        return 0
    lax.fori_loop(0, _NPW // _C1, zcopy, 0)
    pltpu.sync_copy(p1_v.at[0, pl.ds(0, _NPW - (_NPW // _C1) * _C1)],
                    accum.at[pl.ds(row0 + (_NPW // _C1) * _C1,
                                   _NPW - (_NPW // _C1) * _C1)])
    plsc.subcore_barrier()

    def _base(c):
        return wid * _EPW + c * _C1

    def issue_loads(c, j):
        b = _base(c)
        pltpu.async_copy(r_hbm.at[pl.ds(b, _C1)], r_v.at[j], lsem.at[j])
        pltpu.async_copy(edges_hbm.at[pl.ds(b, _C1)], p1_v.at[j], lsem.at[j])

    def wait_loads(c, j):
        b = _base(c)
        pltpu.make_async_copy(r_hbm.at[pl.ds(b, _C1)], r_v.at[j],
                              lsem.at[j]).wait()
        pltpu.make_async_copy(edges_hbm.at[pl.ds(b, _C1)], p1_v.at[j],
                              lsem.at[j]).wait()

    def issue_scatter(j):
        pltpu.async_copy(p1_v.at[j], accum.at[r_v.at[j]], asem.at[j],
                         add=True)

    def wait_scatter(j):
        pltpu.make_async_copy(p1_v.at[j], accum.at[r_v.at[j]],
                              asem.at[j]).wait()

    issue_loads(0, 0)
    issue_loads(1, 1)

    def outer(k, _):
        for jj in range(_NB):
            c = k * _NB + jj
            j = jj
            jl = (jj + 2) % _NB

            @pl.when(c >= 3)
            def _():
                wait_scatter(jl)

            @pl.when(c + 2 < _NCHUNK1)
            def _():
                issue_loads(c + 2, jl)

            wait_loads(c, j)
            issue_scatter(j)
        return 0
    lax.fori_loop(0, _NCHUNK1 // _NB, outer, 0)
    for c in (_NCHUNK1 - 3, _NCHUNK1 - 2, _NCHUNK1 - 1):
        wait_scatter(c % _NB)

    plsc.subcore_barrier()
    _dump_accum(accum, se_hbm, sid, cid)


_SC_CP = pltpu.CompilerParams()
if "needs_layout_passes" in pltpu.CompilerParams.__dataclass_fields__:
    _SC_CP = dataclasses.replace(_SC_CP, needs_layout_passes=False)

_SC_MESH = plsc.VectorSubcoreMesh(core_axis_name="c", subcore_axis_name="s")


@functools.partial(
    pl.kernel,
    compiler_params=_SC_CP,
    out_type=[
        jax.ShapeDtypeStruct((E, D), jnp.float32),
        jax.ShapeDtypeStruct((2 * N, D), jnp.float32),
    ],
    mesh=_SC_MESH,
    scratch_types=[
        pltpu.VMEM((_NBI, _C2), jnp.int32),
        pltpu.VMEM((_NBI, _C2), jnp.int32),
        pltpu.VMEM((_NBD, _C2, D), jnp.float32),
        pltpu.VMEM((_NBD, _C2, D), jnp.float32),
        pltpu.VMEM_SHARED((N, D), jnp.float32),
        pltpu.SemaphoreType.DMA((_NBI,)),
        pltpu.SemaphoreType.DMA((_NBD,)),
        pltpu.SemaphoreType.DMA((_NBD,)),
        pltpu.SemaphoreType.DMA((_NBD,)),
    ],
)
def _sc_gather(*args):
    _sc_gather_body(*args)


@functools.partial(
    pl.kernel,
    compiler_params=_SC_CP,
    out_type=jax.ShapeDtypeStruct((2 * N, D), jnp.float32),
    mesh=_SC_MESH,
    scratch_types=[
        pltpu.VMEM((_NB, _C1), jnp.int32),
        pltpu.VMEM((_NB, _C1, D), jnp.float32),
        pltpu.VMEM_SHARED((N, D), jnp.float32),
        pltpu.SemaphoreType.DMA((_NB,)),
        pltpu.SemaphoreType.DMA((_NB,)),
    ],
)
def _sc_segsum(*args):
    _sc_segsum_body(*args)


def kernel(nodes, edges, senders, receivers, W_e, b_e, W_n, b_n,
           gamma_n, beta_n, gamma_e, beta_e):
    W1, W2, W3 = W_e[:D], W_e[D:2 * D], W_e[2 * D:]
    b_e2 = b_e.reshape(1, D)
    g_e2, bt_e2 = gamma_e.reshape(1, D), beta_e.reshape(1, D)
    g_n2, bt_n2 = gamma_n.reshape(1, D), beta_n.reshape(1, D)

    p1, p2 = pl.pallas_call(
        _proj_body,
        out_shape=[jax.ShapeDtypeStruct((N, D), jnp.float32)] * 2,
    )(nodes, W2, W3, b_e2)

    gp, rg2 = _sc_gather(p1, p2, senders, receivers)
    se2 = _sc_segsum(edges, receivers, rg2)

    BE = 4000
    edges_out = pl.pallas_call(
        _edge_ep_body,
        grid=(E // BE,),
        in_specs=[
            pl.BlockSpec((BE, D), lambda i: (i, 0)),
            pl.BlockSpec((BE, D), lambda i: (i, 0)),
            pl.BlockSpec((D, D), lambda i: (0, 0)),
            pl.BlockSpec((1, D), lambda i: (0, 0)),
            pl.BlockSpec((1, D), lambda i: (0, 0)),
        ],
        out_specs=pl.BlockSpec((BE, D), lambda i: (i, 0)),
        out_shape=jax.ShapeDtypeStruct((E, D), jnp.float32),
    )(edges, gp, W1, g_e2, bt_e2)

    nodes_out = pl.pallas_call(
        _node_body,
        out_shape=jax.ShapeDtypeStruct((N, D), jnp.float32),
    )(nodes, se2, rg2, W1, W_n[:D], W_n[D:], b_n.reshape(1, D), g_n2, bt_n2)

    return nodes_out, edges_out


# speedup vs baseline: 7.0676x; 1.0207x over previous
import dataclasses
import functools

import jax
import jax.numpy as jnp
from jax import lax
from jax.experimental import pallas as pl
from jax.experimental.pallas import tpu as pltpu
from jax.experimental.pallas import tpu_sc as plsc

N = 10000
E = 320000
D = 128

_NC = 2
_NS = 16
_NW = _NC * _NS
_EPW = E // _NW
_C = 16
_NCHUNK = _EPW // _C
_NB = 5
_NPW = N // _NS
_C1 = 40
_NCHUNK1 = _EPW // _C1
_C2 = 40
_NCHUNK2 = _EPW // _C2
_NBD = 4
_NBI = 8


def _proj_body(nodes_ref, w2_ref, w3_ref, be_ref, p1_ref, p2_ref):
    n = nodes_ref[...]
    p1_ref[...] = jnp.dot(
        n, w2_ref[...], preferred_element_type=jnp.float32
    )
    p2_ref[...] = (
        jnp.dot(n, w3_ref[...], preferred_element_type=jnp.float32)
        + be_ref[...]
    )


def _ln(y, g, b):
    mean = jnp.mean(y, axis=-1, keepdims=True)
    yc = y - mean
    var = jnp.mean(yc * yc, axis=-1, keepdims=True)
    return yc * lax.rsqrt(var + 1e-6) * g + b


def _edge_ep_body(e_ref, gp_ref, w1_ref, g_ref, b_ref, out_ref):
    e = e_ref[...]
    ne = (
        jnp.dot(e, w1_ref[...], preferred_element_type=jnp.float32)
        + gp_ref[...]
    )
    out_ref[...] = _ln(ne + e, g_ref[...], b_ref[...])


def _node_body(nodes_ref, se_ref, rg_ref, w1_ref, wn1_ref, wn2_ref, bn_ref,
               g_ref, b_ref, out_ref):
    nodes = nodes_ref[...]
    se = se_ref[:N, :] + se_ref[N:, :]
    r = (
        jnp.dot(se, w1_ref[...], preferred_element_type=jnp.float32)
        + rg_ref[:N, :] + rg_ref[N:, :]
    )
    nn = (
        jnp.dot(nodes, wn1_ref[...], preferred_element_type=jnp.float32)
        + jnp.dot(r, wn2_ref[...], preferred_element_type=jnp.float32)
        + bn_ref[...]
    )
    out_ref[...] = _ln(nn + nodes, g_ref[...], b_ref[...])


def _zero_accum(p1_v, accum, row0):
    def zfill(i, _):
        for g in range(D // 16):
            p1_v[0, i, pl.ds(g * 16, 16)] = jnp.zeros((16,), jnp.float32)
        return 0
    lax.fori_loop(0, _C, zfill, 0)

    def zcopy(k, _):
        pltpu.sync_copy(p1_v.at[0], accum.at[pl.ds(row0 + k * _C, _C)])
        return 0
    lax.fori_loop(0, _NPW // _C, zcopy, 0)
    pltpu.sync_copy(p1_v.at[0, pl.ds(0, 1)],
                    accum.at[pl.ds(row0 + (_NPW // _C) * _C, 1)])


def _dump_accum(accum, dst_hbm, sid, cid):
    o = sid * 624

    @pl.when(sid < _NS - 1)
    def _():
        pltpu.sync_copy(accum.at[pl.ds(o, 624)],
                        dst_hbm.at[pl.ds(cid * N + o, 624)])

    @pl.when(sid == _NS - 1)
    def _():
        pltpu.sync_copy(accum.at[pl.ds(o, 640)],
                        dst_hbm.at[pl.ds(cid * N + o, 640)])


def _sc_gather_body(p1_hbm, p2_hbm, s_hbm, r_hbm,
                    g_hbm, rg_hbm,
                    s_v, r_v, p1_v, p2_v, accum,
                    lsem, gsem, wsem, asem):
    cid = lax.axis_index("c")
    sid = lax.axis_index("s")
    wid = sid * _NC + cid
    row0 = sid * _NPW

    def zfill(i, _):
        for g in range(D // 16):
            p1_v[0, i, pl.ds(g * 16, 16)] = jnp.zeros((16,), jnp.float32)
        return 0
    lax.fori_loop(0, _C2, zfill, 0)

    def zcopy(k, _):
        pltpu.sync_copy(p1_v.at[0], accum.at[pl.ds(row0 + k * _C2, _C2)])
        return 0
    lax.fori_loop(0, _NPW // _C2, zcopy, 0)
    pltpu.sync_copy(p1_v.at[0, pl.ds(0, _NPW - (_NPW // _C2) * _C2)],
                    accum.at[pl.ds(row0 + (_NPW // _C2) * _C2,
                                   _NPW - (_NPW // _C2) * _C2)])
    plsc.subcore_barrier()

    def _base(c):
        return wid * _EPW + c * _C2

    def issue_loads(c, j):
        b = _base(c)
        pltpu.async_copy(s_hbm.at[pl.ds(b, _C2)], s_v.at[j], lsem.at[j])
        pltpu.async_copy(r_hbm.at[pl.ds(b, _C2)], r_v.at[j], lsem.at[j])

    def wait_loads(c, j):
        b = _base(c)
        pltpu.make_async_copy(s_hbm.at[pl.ds(b, _C2)], s_v.at[j],
                              lsem.at[j]).wait()
        pltpu.make_async_copy(r_hbm.at[pl.ds(b, _C2)], r_v.at[j],
                              lsem.at[j]).wait()

    def issue_gathers(ji, jd):
        pltpu.async_copy(p1_hbm.at[s_v.at[ji]], p1_v.at[jd], gsem.at[jd])
        pltpu.async_copy(p2_hbm.at[r_v.at[ji]], p2_v.at[jd], gsem.at[jd])

    def wait_gathers(ji, jd):
        pltpu.make_async_copy(p1_hbm.at[s_v.at[ji]], p1_v.at[jd],
                              gsem.at[jd]).wait()
        pltpu.make_async_copy(p2_hbm.at[r_v.at[ji]], p2_v.at[jd],
                              gsem.at[jd]).wait()

    def issue_writes(c, ji, jd):
        b = _base(c)
        pltpu.async_copy(p1_v.at[jd], g_hbm.at[pl.ds(b, _C2)], wsem.at[jd])
        pltpu.async_copy(p1_v.at[jd], accum.at[r_v.at[ji]], asem.at[jd],
                         add=True)

    def wait_writes(c, ji, jd):
        b = _base(c)
        pltpu.make_async_copy(p1_v.at[jd], g_hbm.at[pl.ds(b, _C2)],
                              wsem.at[jd]).wait()
        pltpu.make_async_copy(p1_v.at[jd], accum.at[r_v.at[ji]],
                              asem.at[jd]).wait()

    def compute(jd):
        p1s, p2s = p1_v.at[jd], p2_v.at[jd]

        def row(i, _):
            for g in range(D // 16):
                sl = pl.ds(g * 16, 16)
                p1s[i, sl] = p1s[i, sl] + p2s[i, sl]
            return 0
        lax.fori_loop(0, _C2, row, 0)

    for c in range(4):
        issue_loads(c, c)
    wait_loads(0, 0)
    issue_gathers(0, 0)
    wait_loads(1, 1)
    issue_gathers(1, 1)

    _MAIN = _NCHUNK2 - (_NCHUNK2 % _NBI)

    def outer(k, _):
        for jj in range(_NBI):
            c = k * _NBI + jj
            ji = jj
            jd = jj % _NBD
            jdg = (jj + 2) % _NBD
            jig = (jj + 2) % _NBI
            jiw = (jj - 2) % _NBI
            jil = (jj + 4) % _NBI

            @pl.when(c >= 2)
            def _():
                wait_writes(c - 2, jiw, jdg)

            @pl.when(c + 4 < _NCHUNK2)
            def _():
                issue_loads(c + 4, jil)

            @pl.when(c + 2 < _NCHUNK2)
            def _():
                wait_loads(c + 2, jig)
                issue_gathers(jig, jdg)

            wait_gathers(ji, jd)
            compute(jd)
            issue_writes(c, ji, jd)
        return 0
    lax.fori_loop(0, _MAIN // _NBI, outer, 0)

    for c in range(_MAIN, _NCHUNK2):
        wait_writes(c - 2, (c - 2) % _NBI, (c - 2) % _NBD)
        wait_gathers(c % _NBI, c % _NBD)
        compute(c % _NBD)
        issue_writes(c, c % _NBI, c % _NBD)
    for c in (_NCHUNK2 - 2, _NCHUNK2 - 1):
        wait_writes(c, c % _NBI, c % _NBD)

    plsc.subcore_barrier()
    _dump_accum(accum, rg_hbm, sid, cid)


def _sc_segsum_body(edges_hbm, r_hbm, tok_hbm,
                    se_hbm,
                    r_v, p1_v, accum, lsem, asem):
    cid = lax.axis_index("c")
    sid = lax.axis_index("s")
    wid = sid * _NC + cid
    row0 = sid * _NPW

    def zfill(i, _):
        for g in range(D // 16):
            p1_v[0, i, pl.ds(g * 16, 16)] = jnp.zeros((16,), jnp.float32)
        return 0
    lax.fori_loop(0, _C1, zfill, 0)

    def zcopy(k, _):
        pltpu.sync_copy(p1_v.at[0], accum.at[pl.ds(row0 + k * _C1, _C1)])
        return 0
    lax.fori_loop(0, _NPW // _C1, zcopy, 0)
    pltpu.sync_copy(p1_v.at[0, pl.ds(0, _NPW - (_NPW // _C1) * _C1)],
                    accum.at[pl.ds(row0 + (_NPW // _C1) * _C1,
                                   _NPW - (_NPW // _C1) * _C1)])
    plsc.subcore_barrier()

    def _base(c):
        return wid * _EPW + c * _C1

    def issue_loads(c, j):
        b = _base(c)
        pltpu.async_copy(r_hbm.at[pl.ds(b, _C1)], r_v.at[j], lsem.at[j])
        pltpu.async_copy(edges_hbm.at[pl.ds(b, _C1)], p1_v.at[j], lsem.at[j])

    def wait_loads(c, j):
        b = _base(c)
        pltpu.make_async_copy(r_hbm.at[pl.ds(b, _C1)], r_v.at[j],
                              lsem.at[j]).wait()
        pltpu.make_async_copy(edges_hbm.at[pl.ds(b, _C1)], p1_v.at[j],
                              lsem.at[j]).wait()

    def issue_scatter(j):
        pltpu.async_copy(p1_v.at[j], accum.at[r_v.at[j]], asem.at[j],
                         add=True)

    def wait_scatter(j):
        pltpu.make_async_copy(p1_v.at[j], accum.at[r_v.at[j]],
                              asem.at[j]).wait()

    issue_loads(0, 0)
    issue_loads(1, 1)

    def outer(k, _):
        for jj in range(_NB):
            c = k * _NB + jj
            j = jj
            jl = (jj + 2) % _NB

            @pl.when(c >= 3)
            def _():
                wait_scatter(jl)

            @pl.when(c + 2 < _NCHUNK1)
            def _():
                issue_loads(c + 2, jl)

            wait_loads(c, j)
            issue_scatter(j)
        return 0
    lax.fori_loop(0, _NCHUNK1 // _NB, outer, 0)
    for c in (_NCHUNK1 - 3, _NCHUNK1 - 2, _NCHUNK1 - 1):
        wait_scatter(c % _NB)

    plsc.subcore_barrier()
    _dump_accum(accum, se_hbm, sid, cid)


_SC_CP = pltpu.CompilerParams()
if "needs_layout_passes" in pltpu.CompilerParams.__dataclass_fields__:
    _SC_CP = dataclasses.replace(_SC_CP, needs_layout_passes=False)

_SC_MESH = plsc.VectorSubcoreMesh(core_axis_name="c", subcore_axis_name="s")


@functools.partial(
    pl.kernel,
    compiler_params=_SC_CP,
    out_type=[
        jax.ShapeDtypeStruct((E, D), jnp.float32),
        jax.ShapeDtypeStruct((2 * N, D), jnp.float32),
    ],
    mesh=_SC_MESH,
    scratch_types=[
        pltpu.VMEM((_NBI, _C2), jnp.int32),
        pltpu.VMEM((_NBI, _C2), jnp.int32),
        pltpu.VMEM((_NBD, _C2, D), jnp.float32),
        pltpu.VMEM((_NBD, _C2, D), jnp.float32),
        pltpu.VMEM_SHARED((N, D), jnp.float32),
        pltpu.SemaphoreType.DMA((_NBI,)),
        pltpu.SemaphoreType.DMA((_NBD,)),
        pltpu.SemaphoreType.DMA((_NBD,)),
        pltpu.SemaphoreType.DMA((_NBD,)),
    ],
)
def _sc_gather(*args):
    _sc_gather_body(*args)


@functools.partial(
    pl.kernel,
    compiler_params=_SC_CP,
    out_type=jax.ShapeDtypeStruct((2 * N, D), jnp.float32),
    mesh=_SC_MESH,
    scratch_types=[
        pltpu.VMEM((_NB, _C1), jnp.int32),
        pltpu.VMEM((_NB, _C1, D), jnp.float32),
        pltpu.VMEM_SHARED((N, D), jnp.float32),
        pltpu.SemaphoreType.DMA((_NB,)),
        pltpu.SemaphoreType.DMA((_NB,)),
    ],
)
def _sc_segsum(*args):
    _sc_segsum_body(*args)


def kernel(nodes, edges, senders, receivers, W_e, b_e, W_n, b_n,
           gamma_n, beta_n, gamma_e, beta_e):
    W1, W2, W3 = W_e[:D], W_e[D:2 * D], W_e[2 * D:]
    b_e2 = b_e.reshape(1, D)
    g_e2, bt_e2 = gamma_e.reshape(1, D), beta_e.reshape(1, D)
    g_n2, bt_n2 = gamma_n.reshape(1, D), beta_n.reshape(1, D)

    p1, p2 = pl.pallas_call(
        _proj_body,
        out_shape=[jax.ShapeDtypeStruct((N, D), jnp.float32)] * 2,
    )(nodes, W2, W3, b_e2)

    gp, rg2 = _sc_gather(p1, p2, senders, receivers)
    se2 = _sc_segsum(edges, receivers, rg2)

    BE = 8000
    edges_out = pl.pallas_call(
        _edge_ep_body,
        grid=(E // BE,),
        in_specs=[
            pl.BlockSpec((BE, D), lambda i: (i, 0)),
            pl.BlockSpec((BE, D), lambda i: (i, 0)),
            pl.BlockSpec((D, D), lambda i: (0, 0)),
            pl.BlockSpec((1, D), lambda i: (0, 0)),
            pl.BlockSpec((1, D), lambda i: (0, 0)),
        ],
        out_specs=pl.BlockSpec((BE, D), lambda i: (i, 0)),
        out_shape=jax.ShapeDtypeStruct((E, D), jnp.float32),
    )(edges, gp, W1, g_e2, bt_e2)

    nodes_out = pl.pallas_call(
        _node_body,
        out_shape=jax.ShapeDtypeStruct((N, D), jnp.float32),
    )(nodes, se2, rg2, W1, W_n[:D], W_n[D:], b_n.reshape(1, D), g_n2, bt_n2)

    return nodes_out, edges_out
